# Initial kernel scaffold; baseline (speedup 1.0000x reference)
import functools

import jax
import jax.numpy as jnp
from jax import lax
from jax.experimental import pallas as pl
from jax.experimental.pallas import tpu as pltpu
from jax.experimental.pallas import tpu_sc as plsc

N = 10000
E = 320000
IN_DIM = 128
HID = 8
HP = 16
NP = 10240
NC, NS = 2, 16
NW = NC * NS
CH = 79
EPT = CH * 128
EPAD = NW * EPT
STR = NP // NS
RB = 1280

_mesh = plsc.VectorSubcoreMesh(core_axis_name="c", subcore_axis_name="s")


def _zero_rows(buf, n_rows):
    def body(i, _):
        buf[i] = jnp.zeros((HP,), jnp.float32)
        return 0
    lax.fori_loop(0, n_rows, body, 0)


@functools.partial(
    pl.kernel,
    out_type=[
        jax.ShapeDtypeStruct((NC, NP, HP), jnp.float32),
        jax.ShapeDtypeStruct((NC, NP, HP), jnp.float32),
    ],
    mesh=_mesh,
    scratch_types=[
        pltpu.VMEM((CH, 128), jnp.int32),
        pltpu.VMEM((CH, 128), jnp.int32),
        pltpu.VMEM((128, HP), jnp.float32),
        pltpu.VMEM((STR, HP), jnp.float32),
        pltpu.VMEM_SHARED((NP, HP), jnp.float32),
        pltpu.VMEM_SHARED((NP, HP), jnp.float32),
    ],
)
def _hist(srcp, dstp, degs_out, degd_out, sidx, didx, ones, obuf, acc_s, acc_d):
    c = lax.axis_index("c")
    s = lax.axis_index("s")
    wid = c * NS + s
    _zero_rows(obuf, STR)
    pltpu.sync_copy(obuf, acc_s.at[pl.ds(s * STR, STR)])
    pltpu.sync_copy(obuf, acc_d.at[pl.ds(s * STR, STR)])

    def fill_ones(i, _):
        ones[i] = jnp.full((HP,), 1.0, jnp.float32)
        return 0
    lax.fori_loop(0, 128, fill_ones, 0)
    pltpu.sync_copy(srcp.at[wid], sidx)
    pltpu.sync_copy(dstp.at[wid], didx)
    plsc.subcore_barrier()

    def step(j, _):
        pltpu.sync_copy(ones, acc_s.at[sidx.at[j]], add=True)
        pltpu.sync_copy(ones, acc_d.at[didx.at[j]], add=True)
        return 0
    lax.fori_loop(0, CH, step, 0)
    plsc.subcore_barrier()
    pltpu.sync_copy(acc_s.at[pl.ds(s * STR, STR)], obuf)
    pltpu.sync_copy(obuf, degs_out.at[c].at[pl.ds(s * STR, STR)])
    pltpu.sync_copy(acc_d.at[pl.ds(s * STR, STR)], obuf)
    pltpu.sync_copy(obuf, degd_out.at[c].at[pl.ds(s * STR, STR)])


@functools.partial(
    pl.kernel,
    out_type=jax.ShapeDtypeStruct((NC, NP, HP), jnp.float32),
    mesh=_mesh,
    scratch_types=[
        pltpu.VMEM((CH, 128), jnp.int32),
        pltpu.VMEM((CH, 128), jnp.int32),
        pltpu.VMEM((128, HP), jnp.float32),
        pltpu.VMEM((STR, HP), jnp.float32),
        pltpu.VMEM_SHARED((NP, HP), jnp.float32),
        pltpu.SemaphoreType.DMA,
    ],
)
def _scat(f_hbm, srcp, dstp, agg_out, sidx, didx, rows, obuf, acc, sem):
    c = lax.axis_index("c")
    s = lax.axis_index("s")
    wid = c * NS + s
    _zero_rows(obuf, STR)
    pltpu.sync_copy(obuf, acc.at[pl.ds(s * STR, STR)])
    pltpu.sync_copy(srcp.at[wid], sidx)
    pltpu.sync_copy(dstp.at[wid], didx)
    plsc.subcore_barrier()

    def step(j, _):
        pltpu.async_copy(f_hbm.at[sidx.at[j]], rows, sem).wait()
        pltpu.sync_copy(rows, acc.at[didx.at[j]], add=True)
        return 0
    lax.fori_loop(0, CH, step, 0)
    plsc.subcore_barrier()
    pltpu.sync_copy(acc.at[pl.ds(s * STR, STR)], obuf)
    pltpu.sync_copy(obuf, agg_out.at[c].at[pl.ds(s * STR, STR)])


def _mm1_body(h_ref, w_ref, o_ref):
    o_ref[...] = jnp.dot(h_ref[...], w_ref[...], preferred_element_type=jnp.float32)


def _mm1(h_pad, w1p):
    return pl.pallas_call(
        _mm1_body,
        grid=(NP // RB,),
        in_specs=[
            pl.BlockSpec((RB, IN_DIM), lambda i: (i, 0)),
            pl.BlockSpec((IN_DIM, HP), lambda i: (0, 0)),
        ],
        out_specs=pl.BlockSpec((RB, HP), lambda i: (i, 0)),
        out_shape=jax.ShapeDtypeStruct((NP, HP), jnp.float32),
    )(h_pad, w1p)


def _prep_body(degs_ref, degd_ref, g1_ref, no_ref, ni_ref, f1_ref):
    i = pl.program_id(0)
    rows = lax.broadcasted_iota(jnp.int32, (RB, HP), 0) + i * RB
    valid = rows < N
    no = jnp.where(valid, lax.rsqrt(degs_ref[0] + degs_ref[1] + 1.0), 0.0)
    ni = jnp.where(valid, lax.rsqrt(degd_ref[0] + degd_ref[1] + 1.0), 0.0)
    no_ref[...] = no
    ni_ref[...] = ni
    f1_ref[...] = g1_ref[...] * no


def _prep(degs, degd, g1):
    spec2 = pl.BlockSpec((NC, RB, HP), lambda i: (0, i, 0))
    spec = pl.BlockSpec((RB, HP), lambda i: (i, 0))
    return pl.pallas_call(
        _prep_body,
        grid=(NP // RB,),
        in_specs=[spec2, spec2, spec],
        out_specs=[spec, spec, spec],
        out_shape=[jax.ShapeDtypeStruct((NP, HP), jnp.float32)] * 3,
    )(degs, degd, g1)


def _mid_body(agg_ref, f1_ref, ni_ref, no_ref, b1_ref, f2_ref):
    a = agg_ref[0] + agg_ref[1] + f1_ref[...]
    h1 = jnp.maximum(a * ni_ref[...] + b1_ref[...], 0.0)
    f2_ref[...] = h1 * no_ref[...]


def _mid(agg1, f1, ni_b, no_b, b1p):
    spec2 = pl.BlockSpec((NC, RB, HP), lambda i: (0, i, 0))
    spec = pl.BlockSpec((RB, HP), lambda i: (i, 0))
    bspec = pl.BlockSpec((1, HP), lambda i: (0, 0))
    return pl.pallas_call(
        _mid_body,
        grid=(NP // RB,),
        in_specs=[spec2, spec, spec, spec, bspec],
        out_specs=spec,
        out_shape=jax.ShapeDtypeStruct((NP, HP), jnp.float32),
    )(agg1, f1, ni_b, no_b, b1p)


def _out_body(agg_ref, f2_ref, ni_ref, b2_ref, w2_ref, wr_ref, br_ref, o_ref):
    a = agg_ref[0] + agg_ref[1] + f2_ref[...]
    h2 = (jnp.dot(a, w2_ref[...], preferred_element_type=jnp.float32)
          * ni_ref[...] + b2_ref[...])
    hr = jnp.maximum(h2, 0.0)
    o_ref[...] = jnp.dot(hr, wr_ref[...], preferred_element_type=jnp.float32) + br_ref[...]


def _outk(agg2, f2, ni_b, b2p, w2p, wrp, brp):
    spec2 = pl.BlockSpec((NC, RB, HP), lambda i: (0, i, 0))
    spec = pl.BlockSpec((RB, HP), lambda i: (i, 0))
    bspec = pl.BlockSpec((1, HP), lambda i: (0, 0))
    return pl.pallas_call(
        _out_body,
        grid=(NP // RB,),
        in_specs=[
            spec2, spec, spec, bspec,
            pl.BlockSpec((HP, HP), lambda i: (0, 0)),
            pl.BlockSpec((HP, IN_DIM), lambda i: (0, 0)),
            pl.BlockSpec((1, IN_DIM), lambda i: (0, 0)),
        ],
        out_specs=pl.BlockSpec((RB, IN_DIM), lambda i: (i, 0)),
        out_shape=jax.ShapeDtypeStruct((NP, IN_DIM), jnp.float32),
    )(agg2, f2, ni_b, b2p, w2p, wrp, brp)


def kernel(h, edge_index, W1, b1, W2, b2, Wr, br):
    src = edge_index[0].astype(jnp.int32)
    dst = edge_index[1].astype(jnp.int32)
    pad = jnp.full((EPAD - E,), N, jnp.int32)
    srcp = jnp.concatenate([src, pad]).reshape(NW, CH, 128)
    dstp = jnp.concatenate([dst, pad]).reshape(NW, CH, 128)
    h_pad = jnp.pad(h, ((0, NP - N), (0, 0)))
    w1p = jnp.pad(W1, ((0, 0), (0, HP - HID)))
    w2p = jnp.pad(W2, ((0, HP - HID), (0, HP - HID)))
    wrp = jnp.pad(Wr, ((0, HP - HID), (0, 0)))
    b1p = jnp.pad(b1, (0, HP - HID)).reshape(1, HP)
    b2p = jnp.pad(b2, (0, HP - HID)).reshape(1, HP)
    brp = br.reshape(1, IN_DIM)

    degs, degd = _hist(srcp, dstp)
    g1 = _mm1(h_pad, w1p)
    no_b, ni_b, f1 = _prep(degs, degd, g1)
    agg1 = _scat(f1, srcp, dstp)
    f2 = _mid(agg1, f1, ni_b, no_b, b1p)
    agg2 = _scat(f2, srcp, dstp)
    out = _outk(agg2, f2, ni_b, b2p, w2p, wrp, brp)
    return out[:N]


# speedup vs baseline: 13.5045x; 13.5045x over previous
import functools

import jax
import jax.numpy as jnp
from jax import lax
from jax.experimental import pallas as pl
from jax.experimental.pallas import tpu as pltpu
from jax.experimental.pallas import tpu_sc as plsc

N = 10000
E = 320000
IN_DIM = 128
HID = 8
HP = 16
NP = 10240
NC, NS = 2, 16
NW = NC * NS
CH = 79
EPT = CH * 128
EPAD = NW * EPT
STR = NP // NS
RB = 1280

_mesh = plsc.VectorSubcoreMesh(core_axis_name="c", subcore_axis_name="s")
_sc_params = pltpu.CompilerParams(use_tc_tiling_on_sc=False)


def _zero_rows(buf, n_rows):
    def body(i, _):
        buf[i] = jnp.zeros((HP,), jnp.float32)
        return 0
    lax.fori_loop(0, n_rows, body, 0)


@functools.partial(
    pl.kernel,
    out_type=[
        jax.ShapeDtypeStruct((NC, NP, HP), jnp.float32),
        jax.ShapeDtypeStruct((NC, NP, HP), jnp.float32),
    ],
    mesh=_mesh,
    scratch_types=[
        pltpu.VMEM((CH, 128), jnp.int32),
        pltpu.VMEM((CH, 128), jnp.int32),
        pltpu.VMEM((128, HP), jnp.float32),
        pltpu.VMEM((STR, HP), jnp.float32),
        pltpu.VMEM_SHARED((NP, HP), jnp.float32),
        pltpu.VMEM_SHARED((NP, HP), jnp.float32),
    ],
    compiler_params=_sc_params,
)
def _hist(srcp, dstp, degs_out, degd_out, sidx, didx, ones, obuf, acc_s, acc_d):
    c = lax.axis_index("c")
    s = lax.axis_index("s")
    wid = c * NS + s
    _zero_rows(obuf, STR)
    pltpu.sync_copy(obuf, acc_s.at[pl.ds(s * STR, STR)])
    pltpu.sync_copy(obuf, acc_d.at[pl.ds(s * STR, STR)])

    def fill_ones(i, _):
        ones[i] = jnp.full((HP,), 1.0, jnp.float32)
        return 0
    lax.fori_loop(0, 128, fill_ones, 0)
    pltpu.sync_copy(srcp.at[wid], sidx)
    pltpu.sync_copy(dstp.at[wid], didx)
    plsc.subcore_barrier()

    def step(j, _):
        pltpu.sync_copy(ones, acc_s.at[sidx.at[j]], add=True)
        pltpu.sync_copy(ones, acc_d.at[didx.at[j]], add=True)
        return 0
    lax.fori_loop(0, CH, step, 0)
    plsc.subcore_barrier()
    pltpu.sync_copy(acc_s.at[pl.ds(s * STR, STR)], obuf)
    pltpu.sync_copy(obuf, degs_out.at[c].at[pl.ds(s * STR, STR)])
    pltpu.sync_copy(acc_d.at[pl.ds(s * STR, STR)], obuf)
    pltpu.sync_copy(obuf, degd_out.at[c].at[pl.ds(s * STR, STR)])


@functools.partial(
    pl.kernel,
    out_type=jax.ShapeDtypeStruct((NC, NP, HP), jnp.float32),
    mesh=_mesh,
    scratch_types=[
        pltpu.VMEM((CH, 128), jnp.int32),
        pltpu.VMEM((CH, 128), jnp.int32),
        pltpu.VMEM((128, HP), jnp.float32),
        pltpu.VMEM((STR, HP), jnp.float32),
        pltpu.VMEM_SHARED((NP, HP), jnp.float32),
        pltpu.SemaphoreType.DMA,
    ],
    compiler_params=_sc_params,
)
def _scat(f_hbm, srcp, dstp, agg_out, sidx, didx, rows, obuf, acc, sem):
    c = lax.axis_index("c")
    s = lax.axis_index("s")
    wid = c * NS + s
    _zero_rows(obuf, STR)
    pltpu.sync_copy(obuf, acc.at[pl.ds(s * STR, STR)])
    pltpu.sync_copy(srcp.at[wid], sidx)
    pltpu.sync_copy(dstp.at[wid], didx)
    plsc.subcore_barrier()

    def step(j, _):
        pltpu.async_copy(f_hbm.at[sidx.at[j]], rows, sem).wait()
        pltpu.sync_copy(rows, acc.at[didx.at[j]], add=True)
        return 0
    lax.fori_loop(0, CH, step, 0)
    plsc.subcore_barrier()
    pltpu.sync_copy(acc.at[pl.ds(s * STR, STR)], obuf)
    pltpu.sync_copy(obuf, agg_out.at[c].at[pl.ds(s * STR, STR)])


def _mm1_body(h_ref, w_ref, o_ref):
    o_ref[...] = jnp.dot(h_ref[...], w_ref[...], preferred_element_type=jnp.float32)


def _mm1(h_pad, w1p):
    return pl.pallas_call(
        _mm1_body,
        grid=(NP // RB,),
        in_specs=[
            pl.BlockSpec((RB, IN_DIM), lambda i: (i, 0)),
            pl.BlockSpec((IN_DIM, HP), lambda i: (0, 0)),
        ],
        out_specs=pl.BlockSpec((RB, HP), lambda i: (i, 0)),
        out_shape=jax.ShapeDtypeStruct((NP, HP), jnp.float32),
    )(h_pad, w1p)


def _prep_body(degs_ref, degd_ref, g1_ref, no_ref, ni_ref, f1_ref):
    i = pl.program_id(0)
    rows = lax.broadcasted_iota(jnp.int32, (RB, HP), 0) + i * RB
    valid = rows < N
    no = jnp.where(valid, lax.rsqrt(degs_ref[0] + degs_ref[1] + 1.0), 0.0)
    ni = jnp.where(valid, lax.rsqrt(degd_ref[0] + degd_ref[1] + 1.0), 0.0)
    no_ref[...] = no
    ni_ref[...] = ni
    f1_ref[...] = g1_ref[...] * no


def _prep(degs, degd, g1):
    spec2 = pl.BlockSpec((NC, RB, HP), lambda i: (0, i, 0))
    spec = pl.BlockSpec((RB, HP), lambda i: (i, 0))
    return pl.pallas_call(
        _prep_body,
        grid=(NP // RB,),
        in_specs=[spec2, spec2, spec],
        out_specs=[spec, spec, spec],
        out_shape=[jax.ShapeDtypeStruct((NP, HP), jnp.float32)] * 3,
    )(degs, degd, g1)


def _mid_body(agg_ref, f1_ref, ni_ref, no_ref, b1_ref, f2_ref):
    a = agg_ref[0] + agg_ref[1] + f1_ref[...]
    h1 = jnp.maximum(a * ni_ref[...] + b1_ref[...], 0.0)
    f2_ref[...] = h1 * no_ref[...]


def _mid(agg1, f1, ni_b, no_b, b1p):
    spec2 = pl.BlockSpec((NC, RB, HP), lambda i: (0, i, 0))
    spec = pl.BlockSpec((RB, HP), lambda i: (i, 0))
    bspec = pl.BlockSpec((1, HP), lambda i: (0, 0))
    return pl.pallas_call(
        _mid_body,
        grid=(NP // RB,),
        in_specs=[spec2, spec, spec, spec, bspec],
        out_specs=spec,
        out_shape=jax.ShapeDtypeStruct((NP, HP), jnp.float32),
    )(agg1, f1, ni_b, no_b, b1p)


def _out_body(agg_ref, f2_ref, ni_ref, b2_ref, w2_ref, wr_ref, br_ref, o_ref):
    a = agg_ref[0] + agg_ref[1] + f2_ref[...]
    h2 = (jnp.dot(a, w2_ref[...], preferred_element_type=jnp.float32)
          * ni_ref[...] + b2_ref[...])
    hr = jnp.maximum(h2, 0.0)
    o_ref[...] = jnp.dot(hr, wr_ref[...], preferred_element_type=jnp.float32) + br_ref[...]


def _outk(agg2, f2, ni_b, b2p, w2p, wrp, brp):
    spec2 = pl.BlockSpec((NC, RB, HP), lambda i: (0, i, 0))
    spec = pl.BlockSpec((RB, HP), lambda i: (i, 0))
    bspec = pl.BlockSpec((1, HP), lambda i: (0, 0))
    return pl.pallas_call(
        _out_body,
        grid=(NP // RB,),
        in_specs=[
            spec2, spec, spec, bspec,
            pl.BlockSpec((HP, HP), lambda i: (0, 0)),
            pl.BlockSpec((HP, IN_DIM), lambda i: (0, 0)),
            pl.BlockSpec((1, IN_DIM), lambda i: (0, 0)),
        ],
        out_specs=pl.BlockSpec((RB, IN_DIM), lambda i: (i, 0)),
        out_shape=jax.ShapeDtypeStruct((NP, IN_DIM), jnp.float32),
    )(agg2, f2, ni_b, b2p, w2p, wrp, brp)


def kernel(h, edge_index, W1, b1, W2, b2, Wr, br):
    src = edge_index[0].astype(jnp.int32)
    dst = edge_index[1].astype(jnp.int32)
    pad = jnp.full((EPAD - E,), N, jnp.int32)
    srcp = jnp.concatenate([src, pad]).reshape(NW, CH, 128)
    dstp = jnp.concatenate([dst, pad]).reshape(NW, CH, 128)
    h_pad = jnp.pad(h, ((0, NP - N), (0, 0)))
    w1p = jnp.pad(W1, ((0, 0), (0, HP - HID)))
    w2p = jnp.pad(W2, ((0, HP - HID), (0, HP - HID)))
    wrp = jnp.pad(Wr, ((0, HP - HID), (0, 0)))
    b1p = jnp.pad(b1, (0, HP - HID)).reshape(1, HP)
    b2p = jnp.pad(b2, (0, HP - HID)).reshape(1, HP)
    brp = br.reshape(1, IN_DIM)

    degs, degd = _hist(srcp, dstp)
    g1 = _mm1(h_pad, w1p)
    no_b, ni_b, f1 = _prep(degs, degd, g1)
    agg1 = _scat(f1, srcp, dstp)
    f2 = _mid(agg1, f1, ni_b, no_b, b1p)
    agg2 = _scat(f2, srcp, dstp)
    out = _outk(agg2, f2, ni_b, b2p, w2p, wrp, brp)
    return out[:N]


# speedup vs baseline: 16.0547x; 1.1888x over previous
import functools

import jax
import jax.numpy as jnp
from jax import lax
from jax.experimental import pallas as pl
from jax.experimental.pallas import tpu as pltpu
from jax.experimental.pallas import tpu_sc as plsc

N = 10000
E = 320000
IN_DIM = 128
HID = 8
HP = 16
NP = 10240
NC, NS = 2, 16
NW = NC * NS
CH = 80
EPT = CH * 128
EPAD = NW * EPT
STR = NP // NS
RB = 1280
NBUF = 4

_mesh = plsc.VectorSubcoreMesh(core_axis_name="c", subcore_axis_name="s")
_sc_params = pltpu.CompilerParams(use_tc_tiling_on_sc=False)


def _zero_rows(buf, n_rows):
    def body(i, _):
        buf[i] = jnp.zeros((HP,), jnp.float32)
        return 0
    lax.fori_loop(0, n_rows, body, 0)


@functools.partial(
    pl.kernel,
    out_type=[
        jax.ShapeDtypeStruct((NC, NP, HP), jnp.float32),
        jax.ShapeDtypeStruct((NC, NP, HP), jnp.float32),
    ],
    mesh=_mesh,
    scratch_types=[
        pltpu.VMEM((CH, 128), jnp.int32),
        pltpu.VMEM((CH, 128), jnp.int32),
        pltpu.VMEM((128, HP), jnp.float32),
        pltpu.VMEM((STR, HP), jnp.float32),
        pltpu.VMEM_SHARED((NP, HP), jnp.float32),
        pltpu.VMEM_SHARED((NP, HP), jnp.float32),
    ],
    compiler_params=_sc_params,
)
def _hist(srcp, dstp, degs_out, degd_out, sidx, didx, ones, obuf, acc_s, acc_d):
    c = lax.axis_index("c")
    s = lax.axis_index("s")
    wid = c * NS + s
    _zero_rows(obuf, STR)
    pltpu.sync_copy(obuf, acc_s.at[pl.ds(s * STR, STR)])
    pltpu.sync_copy(obuf, acc_d.at[pl.ds(s * STR, STR)])

    def fill_ones(i, _):
        ones[i] = jnp.full((HP,), 1.0, jnp.float32)
        return 0
    lax.fori_loop(0, 128, fill_ones, 0)
    pltpu.sync_copy(srcp.at[wid], sidx)
    pltpu.sync_copy(dstp.at[wid], didx)
    plsc.subcore_barrier()

    def step(j, _):
        pltpu.sync_copy(ones, acc_s.at[sidx.at[j]], add=True)
        pltpu.sync_copy(ones, acc_d.at[didx.at[j]], add=True)
        return 0
    lax.fori_loop(0, CH, step, 0)
    plsc.subcore_barrier()
    pltpu.sync_copy(acc_s.at[pl.ds(s * STR, STR)], obuf)
    pltpu.sync_copy(obuf, degs_out.at[c].at[pl.ds(s * STR, STR)])
    pltpu.sync_copy(acc_d.at[pl.ds(s * STR, STR)], obuf)
    pltpu.sync_copy(obuf, degd_out.at[c].at[pl.ds(s * STR, STR)])


@functools.partial(
    pl.kernel,
    out_type=jax.ShapeDtypeStruct((NC, NP, HP), jnp.float32),
    mesh=_mesh,
    scratch_types=[
        pltpu.VMEM((CH, 128), jnp.int32),
        pltpu.VMEM((CH, 128), jnp.int32),
        pltpu.VMEM((NBUF, 128, HP), jnp.float32),
        pltpu.VMEM((STR, HP), jnp.float32),
        pltpu.VMEM_SHARED((NP, HP), jnp.float32),
    ] + [pltpu.SemaphoreType.DMA] * NBUF,
    compiler_params=_sc_params,
)
def _scat(f_hbm, srcp, dstp, agg_out, sidx, didx, rows, obuf, acc, *sems):
    c = lax.axis_index("c")
    s = lax.axis_index("s")
    wid = c * NS + s
    _zero_rows(obuf, STR)
    pltpu.sync_copy(obuf, acc.at[pl.ds(s * STR, STR)])
    pltpu.sync_copy(srcp.at[wid], sidx)
    pltpu.sync_copy(dstp.at[wid], didx)
    plsc.subcore_barrier()

    for b in range(NBUF - 1):
        pltpu.async_copy(f_hbm.at[sidx.at[b]], rows.at[b], sems[b])

    def group(gi, _):
        for b in range(NBUF):
            j = gi * NBUF + b
            jn = j + (NBUF - 1)
            bn = (b + NBUF - 1) % NBUF

            @pl.when(jn < CH)
            def _():
                pltpu.async_copy(f_hbm.at[sidx.at[jn]], rows.at[bn], sems[bn])

            pltpu.make_async_copy(f_hbm.at[sidx.at[j]], rows.at[b], sems[b]).wait()
            pltpu.sync_copy(rows.at[b], acc.at[didx.at[j]], add=True)
        return 0
    lax.fori_loop(0, CH // NBUF, group, 0)
    plsc.subcore_barrier()
    pltpu.sync_copy(acc.at[pl.ds(s * STR, STR)], obuf)
    pltpu.sync_copy(obuf, agg_out.at[c].at[pl.ds(s * STR, STR)])


def _prep_body(degs_ref, degd_ref, h_ref, w_ref, no_ref, ni_ref, f1_ref):
    i = pl.program_id(0)
    rows = lax.broadcasted_iota(jnp.int32, (RB, HP), 0) + i * RB
    valid = rows < N
    no = jnp.where(valid, lax.rsqrt(degs_ref[0] + degs_ref[1] + 1.0), 0.0)
    ni = jnp.where(valid, lax.rsqrt(degd_ref[0] + degd_ref[1] + 1.0), 0.0)
    no_ref[...] = no
    ni_ref[...] = ni
    g = jnp.dot(h_ref[...], w_ref[...], preferred_element_type=jnp.float32)
    f1_ref[...] = g * no


def _prep(degs, degd, h_pad, w1p):
    spec2 = pl.BlockSpec((NC, RB, HP), lambda i: (0, i, 0))
    spec = pl.BlockSpec((RB, HP), lambda i: (i, 0))
    return pl.pallas_call(
        _prep_body,
        grid=(NP // RB,),
        in_specs=[
            spec2, spec2,
            pl.BlockSpec((RB, IN_DIM), lambda i: (i, 0)),
            pl.BlockSpec((IN_DIM, HP), lambda i: (0, 0)),
        ],
        out_specs=[spec, spec, spec],
        out_shape=[jax.ShapeDtypeStruct((NP, HP), jnp.float32)] * 3,
    )(degs, degd, h_pad, w1p)


def _mid_body(agg_ref, f1_ref, ni_ref, no_ref, b1_ref, f2_ref):
    a = agg_ref[0] + agg_ref[1] + f1_ref[...]
    h1 = jnp.maximum(a * ni_ref[...] + b1_ref[...], 0.0)
    f2_ref[...] = h1 * no_ref[...]


def _mid(agg1, f1, ni_b, no_b, b1p):
    spec2 = pl.BlockSpec((NC, RB, HP), lambda i: (0, i, 0))
    spec = pl.BlockSpec((RB, HP), lambda i: (i, 0))
    bspec = pl.BlockSpec((1, HP), lambda i: (0, 0))
    return pl.pallas_call(
        _mid_body,
        grid=(NP // RB,),
        in_specs=[spec2, spec, spec, spec, bspec],
        out_specs=spec,
        out_shape=jax.ShapeDtypeStruct((NP, HP), jnp.float32),
    )(agg1, f1, ni_b, no_b, b1p)


def _out_body(agg_ref, f2_ref, ni_ref, b2_ref, w2_ref, wr_ref, br_ref, o_ref):
    a = agg_ref[0] + agg_ref[1] + f2_ref[...]
    h2 = (jnp.dot(a, w2_ref[...], preferred_element_type=jnp.float32)
          * ni_ref[...] + b2_ref[...])
    hr = jnp.maximum(h2, 0.0)
    o_ref[...] = jnp.dot(hr, wr_ref[...], preferred_element_type=jnp.float32) + br_ref[...]


def _outk(agg2, f2, ni_b, b2p, w2p, wrp, brp):
    spec2 = pl.BlockSpec((NC, RB, HP), lambda i: (0, i, 0))
    spec = pl.BlockSpec((RB, HP), lambda i: (i, 0))
    bspec = pl.BlockSpec((1, HP), lambda i: (0, 0))
    return pl.pallas_call(
        _out_body,
        grid=(NP // RB,),
        in_specs=[
            spec2, spec, spec, bspec,
            pl.BlockSpec((HP, HP), lambda i: (0, 0)),
            pl.BlockSpec((HP, IN_DIM), lambda i: (0, 0)),
            pl.BlockSpec((1, IN_DIM), lambda i: (0, 0)),
        ],
        out_specs=pl.BlockSpec((RB, IN_DIM), lambda i: (i, 0)),
        out_shape=jax.ShapeDtypeStruct((NP, IN_DIM), jnp.float32),
    )(agg2, f2, ni_b, b2p, w2p, wrp, brp)


def kernel(h, edge_index, W1, b1, W2, b2, Wr, br):
    src = edge_index[0].astype(jnp.int32)
    dst = edge_index[1].astype(jnp.int32)
    pad = jnp.full((EPAD - E,), N, jnp.int32)
    srcp = jnp.concatenate([src, pad]).reshape(NW, CH, 128)
    dstp = jnp.concatenate([dst, pad]).reshape(NW, CH, 128)
    h_pad = jnp.pad(h, ((0, NP - N), (0, 0)))
    w1p = jnp.pad(W1, ((0, 0), (0, HP - HID)))
    w2p = jnp.pad(W2, ((0, HP - HID), (0, HP - HID)))
    wrp = jnp.pad(Wr, ((0, HP - HID), (0, 0)))
    b1p = jnp.pad(b1, (0, HP - HID)).reshape(1, HP)
    b2p = jnp.pad(b2, (0, HP - HID)).reshape(1, HP)
    brp = br.reshape(1, IN_DIM)

    degs, degd = _hist(srcp, dstp)
    no_b, ni_b, f1 = _prep(degs, degd, h_pad, w1p)
    agg1 = _scat(f1, srcp, dstp)
    f2 = _mid(agg1, f1, ni_b, no_b, b1p)
    agg2 = _scat(f2, srcp, dstp)
    out = _outk(agg2, f2, ni_b, b2p, w2p, wrp, brp)
    return out[:N]


# speedup vs baseline: 16.4132x; 1.0223x over previous
import functools

import jax
import jax.numpy as jnp
from jax import lax
from jax.experimental import pallas as pl
from jax.experimental.pallas import tpu as pltpu
from jax.experimental.pallas import tpu_sc as plsc

N = 10000
E = 320000
IN_DIM = 128
HID = 8
HP = 16
NP = 10240
NC, NS = 2, 16
NW = NC * NS
CH = 80
EPT = CH * 128
EPAD = NW * EPT
STR = NP // NS
RB = 1280
NBUF = 8
HG = 4

_mesh = plsc.VectorSubcoreMesh(core_axis_name="c", subcore_axis_name="s")
_sc_params = pltpu.CompilerParams(use_tc_tiling_on_sc=False)


def _zero_rows(buf, n_rows):
    def body(i, _):
        buf[i] = jnp.zeros((HP,), jnp.float32)
        return 0
    lax.fori_loop(0, n_rows, body, 0)


@functools.partial(
    pl.kernel,
    out_type=[
        jax.ShapeDtypeStruct((NC, NP, HP), jnp.float32),
        jax.ShapeDtypeStruct((NC, NP, HP), jnp.float32),
    ],
    mesh=_mesh,
    scratch_types=[
        pltpu.VMEM((CH, 128), jnp.int32),
        pltpu.VMEM((CH, 128), jnp.int32),
        pltpu.VMEM((128, HP), jnp.float32),
        pltpu.VMEM((STR, HP), jnp.float32),
        pltpu.VMEM_SHARED((NP, HP), jnp.float32),
        pltpu.VMEM_SHARED((NP, HP), jnp.float32),
        pltpu.SemaphoreType.DMA,
        pltpu.SemaphoreType.DMA,
    ],
    compiler_params=_sc_params,
)
def _hist(srcp, dstp, degs_out, degd_out, sidx, didx, ones, obuf, acc_s, acc_d,
          sem_s, sem_d):
    c = lax.axis_index("c")
    s = lax.axis_index("s")
    wid = c * NS + s
    _zero_rows(obuf, STR)
    pltpu.sync_copy(obuf, acc_s.at[pl.ds(s * STR, STR)])
    pltpu.sync_copy(obuf, acc_d.at[pl.ds(s * STR, STR)])

    def fill_ones(i, _):
        ones[i] = jnp.full((HP,), 1.0, jnp.float32)
        return 0
    lax.fori_loop(0, 128, fill_ones, 0)
    pltpu.sync_copy(srcp.at[wid], sidx)
    pltpu.sync_copy(dstp.at[wid], didx)
    plsc.subcore_barrier()

    def group(gi, _):
        handles = []
        for k in range(HG):
            j = gi * HG + k
            handles.append(
                pltpu.async_copy(ones, acc_s.at[sidx.at[j]], sem_s, add=True))
            handles.append(
                pltpu.async_copy(ones, acc_d.at[didx.at[j]], sem_d, add=True))
        for hdl in handles:
            hdl.wait()
        return 0
    lax.fori_loop(0, CH // HG, group, 0)
    plsc.subcore_barrier()
    pltpu.sync_copy(acc_s.at[pl.ds(s * STR, STR)], obuf)
    pltpu.sync_copy(obuf, degs_out.at[c].at[pl.ds(s * STR, STR)])
    pltpu.sync_copy(acc_d.at[pl.ds(s * STR, STR)], obuf)
    pltpu.sync_copy(obuf, degd_out.at[c].at[pl.ds(s * STR, STR)])


@functools.partial(
    pl.kernel,
    out_type=jax.ShapeDtypeStruct((NC, NP, HP), jnp.float32),
    mesh=_mesh,
    scratch_types=[
        pltpu.VMEM((CH, 128), jnp.int32),
        pltpu.VMEM((CH, 128), jnp.int32),
        pltpu.VMEM((NBUF, 128, HP), jnp.float32),
        pltpu.VMEM((STR, HP), jnp.float32),
        pltpu.VMEM_SHARED((NP, HP), jnp.float32),
    ] + [pltpu.SemaphoreType.DMA] * (2 * NBUF),
    compiler_params=_sc_params,
)
def _scat(f_hbm, srcp, dstp, agg_out, sidx, didx, rows, obuf, acc, *sems):
    gsems, ssems = sems[:NBUF], sems[NBUF:]
    c = lax.axis_index("c")
    s = lax.axis_index("s")
    wid = c * NS + s
    _zero_rows(obuf, STR)
    pltpu.sync_copy(obuf, acc.at[pl.ds(s * STR, STR)])
    pltpu.sync_copy(srcp.at[wid], sidx)
    pltpu.sync_copy(dstp.at[wid], didx)
    plsc.subcore_barrier()

    for b in range(NBUF - 1):
        pltpu.async_copy(f_hbm.at[sidx.at[b]], rows.at[b], gsems[b])

    def group(gi, _):
        for b in range(NBUF):
            j = gi * NBUF + b
            jn = j + (NBUF - 1)
            bn = (b + NBUF - 1) % NBUF

            @pl.when(jn < CH)
            def _():
                @pl.when(j >= 1)
                def _():
                    pltpu.make_async_copy(
                        rows.at[bn], acc.at[didx.at[jn - NBUF]], ssems[bn]
                    ).wait()
                pltpu.async_copy(f_hbm.at[sidx.at[jn]], rows.at[bn], gsems[bn])

            pltpu.make_async_copy(f_hbm.at[sidx.at[j]], rows.at[b], gsems[b]).wait()
            pltpu.async_copy(rows.at[b], acc.at[didx.at[j]], ssems[b], add=True)
        return 0
    lax.fori_loop(0, CH // NBUF, group, 0)
    for b in range(NBUF):
        pltpu.make_async_copy(
            rows.at[b], acc.at[didx.at[CH - NBUF + b]], ssems[b]).wait()
    plsc.subcore_barrier()
    pltpu.sync_copy(acc.at[pl.ds(s * STR, STR)], obuf)
    pltpu.sync_copy(obuf, agg_out.at[c].at[pl.ds(s * STR, STR)])


def _prep_body(degs_ref, degd_ref, h_ref, w_ref, no_ref, ni_ref, f1_ref):
    i = pl.program_id(0)
    rows = lax.broadcasted_iota(jnp.int32, (RB, HP), 0) + i * RB
    valid = rows < N
    no = jnp.where(valid, lax.rsqrt(degs_ref[0] + degs_ref[1] + 1.0), 0.0)
    ni = jnp.where(valid, lax.rsqrt(degd_ref[0] + degd_ref[1] + 1.0), 0.0)
    no_ref[...] = no
    ni_ref[...] = ni
    g = jnp.dot(h_ref[...], w_ref[...], preferred_element_type=jnp.float32)
    f1_ref[...] = g * no


def _prep(degs, degd, h_pad, w1p):
    spec2 = pl.BlockSpec((NC, RB, HP), lambda i: (0, i, 0))
    spec = pl.BlockSpec((RB, HP), lambda i: (i, 0))
    return pl.pallas_call(
        _prep_body,
        grid=(NP // RB,),
        in_specs=[
            spec2, spec2,
            pl.BlockSpec((RB, IN_DIM), lambda i: (i, 0)),
            pl.BlockSpec((IN_DIM, HP), lambda i: (0, 0)),
        ],
        out_specs=[spec, spec, spec],
        out_shape=[jax.ShapeDtypeStruct((NP, HP), jnp.float32)] * 3,
    )(degs, degd, h_pad, w1p)


def _mid_body(agg_ref, f1_ref, ni_ref, no_ref, b1_ref, f2_ref):
    a = agg_ref[0] + agg_ref[1] + f1_ref[...]
    h1 = jnp.maximum(a * ni_ref[...] + b1_ref[...], 0.0)
    f2_ref[...] = h1 * no_ref[...]


def _mid(agg1, f1, ni_b, no_b, b1p):
    spec2 = pl.BlockSpec((NC, RB, HP), lambda i: (0, i, 0))
    spec = pl.BlockSpec((RB, HP), lambda i: (i, 0))
    bspec = pl.BlockSpec((1, HP), lambda i: (0, 0))
    return pl.pallas_call(
        _mid_body,
        grid=(NP // RB,),
        in_specs=[spec2, spec, spec, spec, bspec],
        out_specs=spec,
        out_shape=jax.ShapeDtypeStruct((NP, HP), jnp.float32),
    )(agg1, f1, ni_b, no_b, b1p)


def _out_body(agg_ref, f2_ref, ni_ref, b2_ref, w2_ref, wr_ref, br_ref, o_ref):
    a = agg_ref[0] + agg_ref[1] + f2_ref[...]
    h2 = (jnp.dot(a, w2_ref[...], preferred_element_type=jnp.float32)
          * ni_ref[...] + b2_ref[...])
    hr = jnp.maximum(h2, 0.0)
    o_ref[...] = jnp.dot(hr, wr_ref[...], preferred_element_type=jnp.float32) + br_ref[...]


def _outk(agg2, f2, ni_b, b2p, w2p, wrp, brp):
    spec2 = pl.BlockSpec((NC, RB, HP), lambda i: (0, i, 0))
    spec = pl.BlockSpec((RB, HP), lambda i: (i, 0))
    bspec = pl.BlockSpec((1, HP), lambda i: (0, 0))
    return pl.pallas_call(
        _out_body,
        grid=(NP // RB,),
        in_specs=[
            spec2, spec, spec, bspec,
            pl.BlockSpec((HP, HP), lambda i: (0, 0)),
            pl.BlockSpec((HP, IN_DIM), lambda i: (0, 0)),
            pl.BlockSpec((1, IN_DIM), lambda i: (0, 0)),
        ],
        out_specs=pl.BlockSpec((RB, IN_DIM), lambda i: (i, 0)),
        out_shape=jax.ShapeDtypeStruct((NP, IN_DIM), jnp.float32),
    )(agg2, f2, ni_b, b2p, w2p, wrp, brp)


def kernel(h, edge_index, W1, b1, W2, b2, Wr, br):
    src = edge_index[0].astype(jnp.int32)
    dst = edge_index[1].astype(jnp.int32)
    pad = jnp.full((EPAD - E,), N, jnp.int32)
    srcp = jnp.concatenate([src, pad]).reshape(NW, CH, 128)
    dstp = jnp.concatenate([dst, pad]).reshape(NW, CH, 128)
    h_pad = jnp.pad(h, ((0, NP - N), (0, 0)))
    w1p = jnp.pad(W1, ((0, 0), (0, HP - HID)))
    w2p = jnp.pad(W2, ((0, HP - HID), (0, HP - HID)))
    wrp = jnp.pad(Wr, ((0, HP - HID), (0, 0)))
    b1p = jnp.pad(b1, (0, HP - HID)).reshape(1, HP)
    b2p = jnp.pad(b2, (0, HP - HID)).reshape(1, HP)
    brp = br.reshape(1, IN_DIM)

    degs, degd = _hist(srcp, dstp)
    no_b, ni_b, f1 = _prep(degs, degd, h_pad, w1p)
    agg1 = _scat(f1, srcp, dstp)
    f2 = _mid(agg1, f1, ni_b, no_b, b1p)
    agg2 = _scat(f2, srcp, dstp)
    out = _outk(agg2, f2, ni_b, b2p, w2p, wrp, brp)
    return out[:N]


# speedup vs baseline: 22.2310x; 1.3545x over previous
import functools

import jax
import jax.numpy as jnp
from jax import lax
from jax.experimental import pallas as pl
from jax.experimental.pallas import tpu as pltpu
from jax.experimental.pallas import tpu_sc as plsc

N = 10000
E = 320000
IN_DIM = 128
HID = 8
HP = 16
NP = 10240
NC, NS = 2, 16
NW = NC * NS
CH = 80
EPT = CH * 128
EPAD = NW * EPT
STR = NP // NS
RB = 1280
NBUF = 8
HG = 4

_mesh = plsc.VectorSubcoreMesh(core_axis_name="c", subcore_axis_name="s")
_sc_params = pltpu.CompilerParams(use_tc_tiling_on_sc=False)


def _zero_rows(buf, n_rows):
    def body(i, _):
        buf[i] = jnp.zeros((HP,), jnp.float32)
        return 0
    lax.fori_loop(0, n_rows, body, 0)


@functools.partial(
    pl.kernel,
    out_type=[
        jax.ShapeDtypeStruct((NC, NP, HP), jnp.float32),
        jax.ShapeDtypeStruct((NC, NP, HP), jnp.float32),
    ],
    mesh=_mesh,
    scratch_types=[
        pltpu.VMEM((CH, 128), jnp.int32),
        pltpu.VMEM((CH, 128), jnp.int32),
        pltpu.VMEM((128, HP), jnp.float32),
        pltpu.VMEM((STR, HP), jnp.float32),
        pltpu.VMEM_SHARED((NP, HP), jnp.float32),
        pltpu.VMEM_SHARED((NP, HP), jnp.float32),
        pltpu.SemaphoreType.DMA,
        pltpu.SemaphoreType.DMA,
    ],
    compiler_params=_sc_params,
)
def _hist(srcp, dstp, degs_out, degd_out, sidx, didx, ones, obuf, acc_s, acc_d,
          sem_s, sem_d):
    c = lax.axis_index("c")
    s = lax.axis_index("s")
    wid = c * NS + s
    _zero_rows(obuf, STR)
    pltpu.sync_copy(obuf, acc_s.at[pl.ds(s * STR, STR)])
    pltpu.sync_copy(obuf, acc_d.at[pl.ds(s * STR, STR)])

    def fill_ones(i, _):
        ones[i] = jnp.full((HP,), 1.0, jnp.float32)
        return 0
    lax.fori_loop(0, 128, fill_ones, 0)
    pltpu.sync_copy(srcp.at[wid], sidx)
    pltpu.sync_copy(dstp.at[wid], didx)
    plsc.subcore_barrier()

    def group(gi, _):
        handles = []
        for k in range(HG):
            j = gi * HG + k
            handles.append(
                pltpu.async_copy(ones, acc_s.at[sidx.at[j]], sem_s, add=True))
            handles.append(
                pltpu.async_copy(ones, acc_d.at[didx.at[j]], sem_d, add=True))
        for hdl in handles:
            hdl.wait()
        return 0
    lax.fori_loop(0, CH // HG, group, 0)
    plsc.subcore_barrier()
    pltpu.sync_copy(acc_s.at[pl.ds(s * STR, STR)], obuf)
    pltpu.sync_copy(obuf, degs_out.at[c].at[pl.ds(s * STR, STR)])
    pltpu.sync_copy(acc_d.at[pl.ds(s * STR, STR)], obuf)
    pltpu.sync_copy(obuf, degd_out.at[c].at[pl.ds(s * STR, STR)])


def _scat_edge_loop(sidx, didx, rows, ftab, acc, gsems, ssems):
    for b in range(NBUF - 1):
        pltpu.async_copy(ftab.at[sidx.at[b]], rows.at[b], gsems[b])

    def group(gi, _):
        for b in range(NBUF):
            j = gi * NBUF + b
            jn = j + (NBUF - 1)
            bn = (b + NBUF - 1) % NBUF

            @pl.when(jn < CH)
            def _():
                @pl.when(j >= 1)
                def _():
                    pltpu.make_async_copy(
                        rows.at[bn], acc.at[didx.at[jn - NBUF]], ssems[bn]
                    ).wait()
                pltpu.async_copy(ftab.at[sidx.at[jn]], rows.at[bn], gsems[bn])

            pltpu.make_async_copy(ftab.at[sidx.at[j]], rows.at[b], gsems[b]).wait()
            pltpu.async_copy(rows.at[b], acc.at[didx.at[j]], ssems[b], add=True)
        return 0
    lax.fori_loop(0, CH // NBUF, group, 0)
    for b in range(NBUF):
        pltpu.make_async_copy(
            rows.at[b], acc.at[didx.at[CH - NBUF + b]], ssems[b]).wait()


_scat_scratch = [
    pltpu.VMEM((CH, 128), jnp.int32),
    pltpu.VMEM((CH, 128), jnp.int32),
    pltpu.VMEM((NBUF, 128, HP), jnp.float32),
    pltpu.VMEM((STR, HP), jnp.float32),
    pltpu.VMEM_SHARED((NP, HP), jnp.float32),
    pltpu.VMEM_SHARED((NP, HP), jnp.float32),
] + [pltpu.SemaphoreType.DMA] * (2 * NBUF)


@functools.partial(
    pl.kernel,
    out_type=jax.ShapeDtypeStruct((NC, NP, HP), jnp.float32),
    mesh=_mesh,
    scratch_types=_scat_scratch,
    compiler_params=_sc_params,
)
def _scat1(f_hbm, srcp, dstp, agg_out, sidx, didx, rows, obuf, ftab, acc, *sems):
    c = lax.axis_index("c")
    s = lax.axis_index("s")
    wid = c * NS + s
    _zero_rows(obuf, STR)
    pltpu.sync_copy(obuf, acc.at[pl.ds(s * STR, STR)])
    pltpu.sync_copy(f_hbm.at[pl.ds(s * STR, STR)], obuf)
    pltpu.sync_copy(obuf, ftab.at[pl.ds(s * STR, STR)])
    pltpu.sync_copy(srcp.at[wid], sidx)
    pltpu.sync_copy(dstp.at[wid], didx)
    plsc.subcore_barrier()
    _scat_edge_loop(sidx, didx, rows, ftab, acc, sems[:NBUF], sems[NBUF:])
    plsc.subcore_barrier()
    pltpu.sync_copy(acc.at[pl.ds(s * STR, STR)], obuf)
    pltpu.sync_copy(obuf, agg_out.at[c].at[pl.ds(s * STR, STR)])


@functools.partial(
    pl.kernel,
    out_type=jax.ShapeDtypeStruct((NC, NP, HP), jnp.float32),
    mesh=_mesh,
    scratch_types=[
        pltpu.VMEM((STR, HP), jnp.float32),
        pltpu.VMEM((STR, HP), jnp.float32),
        pltpu.VMEM((STR, HP), jnp.float32),
        pltpu.VMEM((STR, HP), jnp.float32),
        pltpu.VMEM((STR, HP), jnp.float32),
        pltpu.VMEM((1, HP), jnp.float32),
    ] + _scat_scratch,
    compiler_params=_sc_params,
)
def _scat2(agg1p, f1_hbm, ni_hbm, no_hbm, b1_hbm, srcp, dstp, agg_out,
           a0, a1, v_f1, v_ni, v_no, v_b1,
           sidx, didx, rows, obuf, ftab, acc, *sems):
    c = lax.axis_index("c")
    s = lax.axis_index("s")
    wid = c * NS + s
    _zero_rows(obuf, STR)
    pltpu.sync_copy(obuf, acc.at[pl.ds(s * STR, STR)])
    sl = pl.ds(s * STR, STR)
    pltpu.sync_copy(agg1p.at[0].at[sl], a0)
    pltpu.sync_copy(agg1p.at[1].at[sl], a1)
    pltpu.sync_copy(f1_hbm.at[sl], v_f1)
    pltpu.sync_copy(ni_hbm.at[sl], v_ni)
    pltpu.sync_copy(no_hbm.at[sl], v_no)
    pltpu.sync_copy(b1_hbm, v_b1)
    b1v = v_b1[0]

    def frow(i, _):
        a = a0[i] + a1[i] + v_f1[i]
        h1 = jnp.maximum(a * v_ni[i] + b1v, 0.0)
        obuf[i] = h1 * v_no[i]
        return 0
    lax.fori_loop(0, STR, frow, 0)
    pltpu.sync_copy(obuf, ftab.at[sl])
    pltpu.sync_copy(srcp.at[wid], sidx)
    pltpu.sync_copy(dstp.at[wid], didx)
    plsc.subcore_barrier()
    _scat_edge_loop(sidx, didx, rows, ftab, acc, sems[:NBUF], sems[NBUF:])
    plsc.subcore_barrier()
    pltpu.sync_copy(acc.at[pl.ds(s * STR, STR)], obuf)
    pltpu.sync_copy(obuf, agg_out.at[c].at[pl.ds(s * STR, STR)])


def _prep_body(degs_ref, degd_ref, h_ref, w_ref, no_ref, ni_ref, f1_ref):
    i = pl.program_id(0)
    rows = lax.broadcasted_iota(jnp.int32, (RB, HP), 0) + i * RB
    valid = rows < N
    no = jnp.where(valid, lax.rsqrt(degs_ref[0] + degs_ref[1] + 1.0), 0.0)
    ni = jnp.where(valid, lax.rsqrt(degd_ref[0] + degd_ref[1] + 1.0), 0.0)
    no_ref[...] = no
    ni_ref[...] = ni
    g = jnp.dot(h_ref[...], w_ref[...], preferred_element_type=jnp.float32)
    f1_ref[...] = g * no


def _prep(degs, degd, h_pad, w1p):
    spec2 = pl.BlockSpec((NC, RB, HP), lambda i: (0, i, 0))
    spec = pl.BlockSpec((RB, HP), lambda i: (i, 0))
    return pl.pallas_call(
        _prep_body,
        grid=(NP // RB,),
        in_specs=[
            spec2, spec2,
            pl.BlockSpec((RB, IN_DIM), lambda i: (i, 0)),
            pl.BlockSpec((IN_DIM, HP), lambda i: (0, 0)),
        ],
        out_specs=[spec, spec, spec],
        out_shape=[jax.ShapeDtypeStruct((NP, HP), jnp.float32)] * 3,
    )(degs, degd, h_pad, w1p)


def _out_body(agg1_ref, agg2_ref, f1_ref, ni_ref, no_ref, b1_ref, b2_ref,
              w2_ref, wr_ref, br_ref, o_ref):
    a1 = agg1_ref[0] + agg1_ref[1] + f1_ref[...]
    h1 = jnp.maximum(a1 * ni_ref[...] + b1_ref[...], 0.0)
    f2 = h1 * no_ref[...]
    a2 = agg2_ref[0] + agg2_ref[1] + f2
    h2 = (jnp.dot(a2, w2_ref[...], preferred_element_type=jnp.float32)
          * ni_ref[...] + b2_ref[...])
    hr = jnp.maximum(h2, 0.0)
    o_ref[...] = jnp.dot(hr, wr_ref[...], preferred_element_type=jnp.float32) + br_ref[...]


def _outk(agg1, agg2, f1, ni_b, no_b, b1p, b2p, w2p, wrp, brp):
    spec2 = pl.BlockSpec((NC, RB, HP), lambda i: (0, i, 0))
    spec = pl.BlockSpec((RB, HP), lambda i: (i, 0))
    bspec = pl.BlockSpec((1, HP), lambda i: (0, 0))
    return pl.pallas_call(
        _out_body,
        grid=(NP // RB,),
        in_specs=[
            spec2, spec2, spec, spec, spec, bspec, bspec,
            pl.BlockSpec((HP, HP), lambda i: (0, 0)),
            pl.BlockSpec((HP, IN_DIM), lambda i: (0, 0)),
            pl.BlockSpec((1, IN_DIM), lambda i: (0, 0)),
        ],
        out_specs=pl.BlockSpec((RB, IN_DIM), lambda i: (i, 0)),
        out_shape=jax.ShapeDtypeStruct((NP, IN_DIM), jnp.float32),
    )(agg1, agg2, f1, ni_b, no_b, b1p, b2p, w2p, wrp, brp)


def kernel(h, edge_index, W1, b1, W2, b2, Wr, br):
    src = edge_index[0].astype(jnp.int32)
    dst = edge_index[1].astype(jnp.int32)
    pad = jnp.full((EPAD - E,), N, jnp.int32)
    srcp = jnp.concatenate([src, pad]).reshape(NW, CH, 128)
    dstp = jnp.concatenate([dst, pad]).reshape(NW, CH, 128)
    h_pad = jnp.pad(h, ((0, NP - N), (0, 0)))
    w1p = jnp.pad(W1, ((0, 0), (0, HP - HID)))
    w2p = jnp.pad(W2, ((0, HP - HID), (0, HP - HID)))
    wrp = jnp.pad(Wr, ((0, HP - HID), (0, 0)))
    b1p = jnp.pad(b1, (0, HP - HID)).reshape(1, HP)
    b2p = jnp.pad(b2, (0, HP - HID)).reshape(1, HP)
    brp = br.reshape(1, IN_DIM)

    degs, degd = _hist(srcp, dstp)
    no_b, ni_b, f1 = _prep(degs, degd, h_pad, w1p)
    agg1 = _scat1(f1, srcp, dstp)
    agg2 = _scat2(agg1, f1, ni_b, no_b, b1p, srcp, dstp)
    out = _outk(agg1, agg2, f1, ni_b, no_b, b1p, b2p, w2p, wrp, brp)
    return out[:N]


# speedup vs baseline: 22.3367x; 1.0048x over previous
import functools

import jax
import jax.numpy as jnp
from jax import lax
from jax.experimental import pallas as pl
from jax.experimental.pallas import tpu as pltpu
from jax.experimental.pallas import tpu_sc as plsc

N = 10000
E = 320000
IN_DIM = 128
HID = 8
HP = 16
NP = 10240
NC, NS = 2, 16
NW = NC * NS
CH = 80
EPT = CH * 128
EPAD = NW * EPT
STR = NP // NS
RB = 1280
NBUF = 8
HG = 4

_mesh = plsc.VectorSubcoreMesh(core_axis_name="c", subcore_axis_name="s")
_sc_params = pltpu.CompilerParams(use_tc_tiling_on_sc=False)


def _zero_rows(buf, n_rows):
    def body(i, _):
        buf[i] = jnp.zeros((HP,), jnp.float32)
        return 0
    lax.fori_loop(0, n_rows, body, 0)


@functools.partial(
    pl.kernel,
    out_type=[
        jax.ShapeDtypeStruct((NC, NP, HP), jnp.float32),
        jax.ShapeDtypeStruct((NC, NP, HP), jnp.float32),
    ],
    mesh=_mesh,
    scratch_types=[
        pltpu.VMEM((CH, 128), jnp.int32),
        pltpu.VMEM((CH, 128), jnp.int32),
        pltpu.VMEM((128, HP), jnp.float32),
        pltpu.VMEM((STR, HP), jnp.float32),
        pltpu.VMEM_SHARED((NP, HP), jnp.float32),
        pltpu.VMEM_SHARED((NP, HP), jnp.float32),
        pltpu.SemaphoreType.DMA,
        pltpu.SemaphoreType.DMA,
    ],
    compiler_params=_sc_params,
)
def _hist(srcp, dstp, degs_out, degd_out, sidx, didx, ones, obuf, acc_s, acc_d,
          sem_s, sem_d):
    c = lax.axis_index("c")
    s = lax.axis_index("s")
    wid = c * NS + s
    _zero_rows(obuf, STR)
    pltpu.sync_copy(obuf, acc_s.at[pl.ds(s * STR, STR)])
    pltpu.sync_copy(obuf, acc_d.at[pl.ds(s * STR, STR)])

    def fill_ones(i, _):
        ones[i] = jnp.full((HP,), 1.0, jnp.float32)
        return 0
    lax.fori_loop(0, 128, fill_ones, 0)
    pltpu.sync_copy(srcp.at[wid], sidx)
    pltpu.sync_copy(dstp.at[wid], didx)
    plsc.subcore_barrier()

    def group(gi, _):
        handles = []
        for k in range(HG):
            j = gi * HG + k
            handles.append(
                pltpu.async_copy(ones, acc_s.at[sidx.at[j]], sem_s, add=True))
            handles.append(
                pltpu.async_copy(ones, acc_d.at[didx.at[j]], sem_d, add=True))
        for hdl in handles:
            hdl.wait()
        return 0
    lax.fori_loop(0, CH // HG, group, 0)
    plsc.subcore_barrier()
    pltpu.sync_copy(acc_s.at[pl.ds(s * STR, STR)], obuf)
    pltpu.sync_copy(obuf, degs_out.at[c].at[pl.ds(s * STR, STR)])
    pltpu.sync_copy(acc_d.at[pl.ds(s * STR, STR)], obuf)
    pltpu.sync_copy(obuf, degd_out.at[c].at[pl.ds(s * STR, STR)])


def _scat_edge_loop(sidx, didx, rows, ftab, acc, gsems, ssems):
    for b in range(NBUF - 1):
        pltpu.async_copy(ftab.at[sidx.at[b]], rows.at[b], gsems[b])

    def group(gi, _):
        for b in range(NBUF):
            j = gi * NBUF + b
            jn = j + (NBUF - 1)
            bn = (b + NBUF - 1) % NBUF

            @pl.when(jn < CH)
            def _():
                @pl.when(j >= 1)
                def _():
                    pltpu.make_async_copy(
                        rows.at[bn], acc.at[didx.at[jn - NBUF]], ssems[bn]
                    ).wait()
                pltpu.async_copy(ftab.at[sidx.at[jn]], rows.at[bn], gsems[bn])

            pltpu.make_async_copy(ftab.at[sidx.at[j]], rows.at[b], gsems[b]).wait()
            pltpu.async_copy(rows.at[b], acc.at[didx.at[j]], ssems[b], add=True)
        return 0
    lax.fori_loop(0, CH // NBUF, group, 0)
    for b in range(NBUF):
        pltpu.make_async_copy(
            rows.at[b], acc.at[didx.at[CH - NBUF + b]], ssems[b]).wait()


_scat_scratch = [
    pltpu.VMEM((CH, 128), jnp.int32),
    pltpu.VMEM((CH, 128), jnp.int32),
    pltpu.VMEM((NBUF, 128, HP), jnp.float32),
    pltpu.VMEM((STR, HP), jnp.float32),
    pltpu.VMEM_SHARED((NP, HP), jnp.float32),
    pltpu.VMEM_SHARED((NP, HP), jnp.float32),
] + [pltpu.SemaphoreType.DMA] * (2 * NBUF)


@functools.partial(
    pl.kernel,
    out_type=jax.ShapeDtypeStruct((NC, NP, HP), jnp.float32),
    mesh=_mesh,
    scratch_types=_scat_scratch,
    compiler_params=_sc_params,
)
def _scat1(f_hbm, srcp, dstp, agg_out, sidx, didx, rows, obuf, ftab, acc, *sems):
    c = lax.axis_index("c")
    s = lax.axis_index("s")
    wid = c * NS + s
    _zero_rows(obuf, STR)
    pltpu.sync_copy(obuf, acc.at[pl.ds(s * STR, STR)])
    pltpu.sync_copy(f_hbm.at[pl.ds(s * STR, STR)], obuf)
    pltpu.sync_copy(obuf, ftab.at[pl.ds(s * STR, STR)])
    pltpu.sync_copy(srcp.at[wid], sidx)
    pltpu.sync_copy(dstp.at[wid], didx)
    plsc.subcore_barrier()
    _scat_edge_loop(sidx, didx, rows, ftab, acc, sems[:NBUF], sems[NBUF:])
    plsc.subcore_barrier()
    pltpu.sync_copy(acc.at[pl.ds(s * STR, STR)], obuf)
    pltpu.sync_copy(obuf, agg_out.at[c].at[pl.ds(s * STR, STR)])


@functools.partial(
    pl.kernel,
    out_type=jax.ShapeDtypeStruct((NC, NP, HP), jnp.float32),
    mesh=_mesh,
    scratch_types=[
        pltpu.VMEM((STR, HP), jnp.float32),
        pltpu.VMEM((STR, HP), jnp.float32),
        pltpu.VMEM((STR, HP), jnp.float32),
        pltpu.VMEM((STR, HP), jnp.float32),
        pltpu.VMEM((STR, HP), jnp.float32),
        pltpu.VMEM((1, HP), jnp.float32),
    ] + _scat_scratch,
    compiler_params=_sc_params,
)
def _scat2(agg1p, f1_hbm, ni_hbm, no_hbm, b1_hbm, srcp, dstp, agg_out,
           a0, a1, v_f1, v_ni, v_no, v_b1,
           sidx, didx, rows, obuf, ftab, acc, *sems):
    c = lax.axis_index("c")
    s = lax.axis_index("s")
    wid = c * NS + s
    _zero_rows(obuf, STR)
    pltpu.sync_copy(obuf, acc.at[pl.ds(s * STR, STR)])
    sl = pl.ds(s * STR, STR)
    pltpu.sync_copy(agg1p.at[0].at[sl], a0)
    pltpu.sync_copy(agg1p.at[1].at[sl], a1)
    pltpu.sync_copy(f1_hbm.at[sl], v_f1)
    pltpu.sync_copy(ni_hbm.at[sl], v_ni)
    pltpu.sync_copy(no_hbm.at[sl], v_no)
    pltpu.sync_copy(b1_hbm, v_b1)
    b1v = v_b1[0]

    def frow(i4, _):
        for k in range(4):
            i = i4 * 4 + k
            a = a0[i] + a1[i] + v_f1[i]
            h1 = jnp.maximum(a * v_ni[i] + b1v, 0.0)
            obuf[i] = h1 * v_no[i]
        return 0
    lax.fori_loop(0, STR // 4, frow, 0)
    pltpu.sync_copy(obuf, ftab.at[sl])
    pltpu.sync_copy(srcp.at[wid], sidx)
    pltpu.sync_copy(dstp.at[wid], didx)
    plsc.subcore_barrier()
    _scat_edge_loop(sidx, didx, rows, ftab, acc, sems[:NBUF], sems[NBUF:])
    plsc.subcore_barrier()
    pltpu.sync_copy(acc.at[pl.ds(s * STR, STR)], obuf)
    pltpu.sync_copy(obuf, agg_out.at[c].at[pl.ds(s * STR, STR)])


def _mm1_body(h_ref, w_ref, o_ref):
    o_ref[...] = jnp.dot(h_ref[...], w_ref[...], preferred_element_type=jnp.float32)


def _mm1(h_pad, w1p):
    return pl.pallas_call(
        _mm1_body,
        grid=(NP // RB,),
        in_specs=[
            pl.BlockSpec((RB, IN_DIM), lambda i: (i, 0)),
            pl.BlockSpec((IN_DIM, HP), lambda i: (0, 0)),
        ],
        out_specs=pl.BlockSpec((RB, HP), lambda i: (i, 0)),
        out_shape=jax.ShapeDtypeStruct((NP, HP), jnp.float32),
    )(h_pad, w1p)


def _prep_body(degs_ref, degd_ref, g1_ref, no_ref, ni_ref, f1_ref):
    i = pl.program_id(0)
    rows = lax.broadcasted_iota(jnp.int32, (RB, HP), 0) + i * RB
    valid = rows < N
    no = jnp.where(valid, lax.rsqrt(degs_ref[0] + degs_ref[1] + 1.0), 0.0)
    ni = jnp.where(valid, lax.rsqrt(degd_ref[0] + degd_ref[1] + 1.0), 0.0)
    no_ref[...] = no
    ni_ref[...] = ni
    f1_ref[...] = g1_ref[...] * no


def _prep(degs, degd, g1):
    spec2 = pl.BlockSpec((NC, RB, HP), lambda i: (0, i, 0))
    spec = pl.BlockSpec((RB, HP), lambda i: (i, 0))
    return pl.pallas_call(
        _prep_body,
        grid=(NP // RB,),
        in_specs=[spec2, spec2, spec],
        out_specs=[spec, spec, spec],
        out_shape=[jax.ShapeDtypeStruct((NP, HP), jnp.float32)] * 3,
    )(degs, degd, g1)


def _out_body(agg1_ref, agg2_ref, f1_ref, ni_ref, no_ref, b1_ref, b2_ref,
              w2_ref, wr_ref, br_ref, o_ref):
    a1 = agg1_ref[0] + agg1_ref[1] + f1_ref[...]
    h1 = jnp.maximum(a1 * ni_ref[...] + b1_ref[...], 0.0)
    f2 = h1 * no_ref[...]
    a2 = agg2_ref[0] + agg2_ref[1] + f2
    h2 = (jnp.dot(a2, w2_ref[...], preferred_element_type=jnp.float32)
          * ni_ref[...] + b2_ref[...])
    hr = jnp.maximum(h2, 0.0)
    o_ref[...] = jnp.dot(hr, wr_ref[...], preferred_element_type=jnp.float32) + br_ref[...]


def _outk(agg1, agg2, f1, ni_b, no_b, b1p, b2p, w2p, wrp, brp):
    spec2 = pl.BlockSpec((NC, RB, HP), lambda i: (0, i, 0))
    spec = pl.BlockSpec((RB, HP), lambda i: (i, 0))
    bspec = pl.BlockSpec((1, HP), lambda i: (0, 0))
    return pl.pallas_call(
        _out_body,
        grid=(NP // RB,),
        in_specs=[
            spec2, spec2, spec, spec, spec, bspec, bspec,
            pl.BlockSpec((HP, HP), lambda i: (0, 0)),
            pl.BlockSpec((HP, IN_DIM), lambda i: (0, 0)),
            pl.BlockSpec((1, IN_DIM), lambda i: (0, 0)),
        ],
        out_specs=pl.BlockSpec((RB, IN_DIM), lambda i: (i, 0)),
        out_shape=jax.ShapeDtypeStruct((NP, IN_DIM), jnp.float32),
    )(agg1, agg2, f1, ni_b, no_b, b1p, b2p, w2p, wrp, brp)


def kernel(h, edge_index, W1, b1, W2, b2, Wr, br):
    src = edge_index[0].astype(jnp.int32)
    dst = edge_index[1].astype(jnp.int32)
    pad = jnp.full((EPAD - E,), N, jnp.int32)
    srcp = jnp.concatenate([src, pad]).reshape(NW, CH, 128)
    dstp = jnp.concatenate([dst, pad]).reshape(NW, CH, 128)
    h_pad = jnp.pad(h, ((0, NP - N), (0, 0)))
    w1p = jnp.pad(W1, ((0, 0), (0, HP - HID)))
    w2p = jnp.pad(W2, ((0, HP - HID), (0, HP - HID)))
    wrp = jnp.pad(Wr, ((0, HP - HID), (0, 0)))
    b1p = jnp.pad(b1, (0, HP - HID)).reshape(1, HP)
    b2p = jnp.pad(b2, (0, HP - HID)).reshape(1, HP)
    brp = br.reshape(1, IN_DIM)

    g1 = _mm1(h_pad, w1p)
    degs, degd = _hist(srcp, dstp)
    no_b, ni_b, f1 = _prep(degs, degd, g1)
    agg1 = _scat1(f1, srcp, dstp)
    agg2 = _scat2(agg1, f1, ni_b, no_b, b1p, srcp, dstp)
    out = _outk(agg1, agg2, f1, ni_b, no_b, b1p, b2p, w2p, wrp, brp)
    return out[:N]


# speedup vs baseline: 24.1190x; 1.0798x over previous
import functools

import jax
import jax.numpy as jnp
from jax import lax
from jax.experimental import pallas as pl
from jax.experimental.pallas import tpu as pltpu
from jax.experimental.pallas import tpu_sc as plsc

N = 10000
E = 320000
IN_DIM = 128
HID = 8
HP = 16
NP = 10240
NC, NS = 2, 16
NW = NC * NS
CH = 80
EPT = CH * 128
EPAD = NW * EPT
STR = NP // NS
RB = 1280
NBUF = 8
HG = 4

_mesh = plsc.VectorSubcoreMesh(core_axis_name="c", subcore_axis_name="s")
_sc_params = pltpu.CompilerParams(use_tc_tiling_on_sc=False)


def _zero_rows(buf, n_rows):
    def body(i, _):
        buf[i] = jnp.zeros((HP,), jnp.float32)
        return 0
    lax.fori_loop(0, n_rows, body, 0)


def _qrsqrt(x):
    i = lax.bitcast_convert_type(x, jnp.int32)
    y = lax.bitcast_convert_type(jnp.int32(0x5F3759DF) - (i >> 1), jnp.float32)
    for _ in range(3):
        y = y * (1.5 - 0.5 * x * y * y)
    return y


@functools.partial(
    pl.kernel,
    out_type=[
        jax.ShapeDtypeStruct((NC, NP, HP), jnp.float32),
        jax.ShapeDtypeStruct((NC, NP, HP), jnp.float32),
    ],
    mesh=_mesh,
    scratch_types=[
        pltpu.VMEM((CH, 128), jnp.int32),
        pltpu.VMEM((CH, 128), jnp.int32),
        pltpu.VMEM((128, HP), jnp.float32),
        pltpu.VMEM((STR, HP), jnp.float32),
        pltpu.VMEM_SHARED((NP, HP), jnp.float32),
        pltpu.VMEM_SHARED((NP, HP), jnp.float32),
        pltpu.SemaphoreType.DMA,
        pltpu.SemaphoreType.DMA,
    ],
    compiler_params=_sc_params,
)
def _hist(srcp, dstp, degs_out, degd_out, sidx, didx, ones, obuf, acc_s, acc_d,
          sem_s, sem_d):
    c = lax.axis_index("c")
    s = lax.axis_index("s")
    wid = c * NS + s
    _zero_rows(obuf, STR)
    pltpu.sync_copy(obuf, acc_s.at[pl.ds(s * STR, STR)])
    pltpu.sync_copy(obuf, acc_d.at[pl.ds(s * STR, STR)])

    def fill_ones(i, _):
        ones[i] = jnp.full((HP,), 1.0, jnp.float32)
        return 0
    lax.fori_loop(0, 128, fill_ones, 0)
    pltpu.sync_copy(srcp.at[wid], sidx)
    pltpu.sync_copy(dstp.at[wid], didx)
    plsc.subcore_barrier()

    def group(gi, _):
        handles = []
        for k in range(HG):
            j = gi * HG + k
            handles.append(
                pltpu.async_copy(ones, acc_s.at[sidx.at[j]], sem_s, add=True))
            handles.append(
                pltpu.async_copy(ones, acc_d.at[didx.at[j]], sem_d, add=True))
        for hdl in handles:
            hdl.wait()
        return 0
    lax.fori_loop(0, CH // HG, group, 0)
    plsc.subcore_barrier()
    pltpu.sync_copy(acc_s.at[pl.ds(s * STR, STR)], obuf)
    pltpu.sync_copy(obuf, degs_out.at[c].at[pl.ds(s * STR, STR)])
    pltpu.sync_copy(acc_d.at[pl.ds(s * STR, STR)], obuf)
    pltpu.sync_copy(obuf, degd_out.at[c].at[pl.ds(s * STR, STR)])


def _scat_edge_loop(sidx, didx, rows, ftab, acc, gsems, ssems):
    for b in range(NBUF - 1):
        pltpu.async_copy(ftab.at[sidx.at[b]], rows.at[b], gsems[b])

    def group(gi, _):
        for b in range(NBUF):
            j = gi * NBUF + b
            jn = j + (NBUF - 1)
            bn = (b + NBUF - 1) % NBUF

            @pl.when(jn < CH)
            def _():
                @pl.when(j >= 1)
                def _():
                    pltpu.make_async_copy(
                        rows.at[bn], acc.at[didx.at[jn - NBUF]], ssems[bn]
                    ).wait()
                pltpu.async_copy(ftab.at[sidx.at[jn]], rows.at[bn], gsems[bn])

            pltpu.make_async_copy(ftab.at[sidx.at[j]], rows.at[b], gsems[b]).wait()
            pltpu.async_copy(rows.at[b], acc.at[didx.at[j]], ssems[b], add=True)
        return 0
    lax.fori_loop(0, CH // NBUF, group, 0)
    for b in range(NBUF):
        pltpu.make_async_copy(
            rows.at[b], acc.at[didx.at[CH - NBUF + b]], ssems[b]).wait()


_scat_scratch = [
    pltpu.VMEM((CH, 128), jnp.int32),
    pltpu.VMEM((CH, 128), jnp.int32),
    pltpu.VMEM((NBUF, 128, HP), jnp.float32),
    pltpu.VMEM((STR, HP), jnp.float32),
    pltpu.VMEM_SHARED((NP, HP), jnp.float32),
    pltpu.VMEM_SHARED((NP, HP), jnp.float32),
] + [pltpu.SemaphoreType.DMA] * (2 * NBUF)


@functools.partial(
    pl.kernel,
    out_type=jax.ShapeDtypeStruct((NC, NP, HP), jnp.float32),
    mesh=_mesh,
    scratch_types=[
        pltpu.VMEM((STR, HP), jnp.float32),
        pltpu.VMEM((STR, HP), jnp.float32),
        pltpu.VMEM((STR, HP), jnp.float32),
    ] + _scat_scratch,
    compiler_params=_sc_params,
)
def _scat1(g1_hbm, degs, srcp, dstp, agg_out,
           d0, d1, vg1, sidx, didx, rows, obuf, ftab, acc, *sems):
    c = lax.axis_index("c")
    s = lax.axis_index("s")
    wid = c * NS + s
    sl = pl.ds(s * STR, STR)
    _zero_rows(obuf, STR)
    pltpu.sync_copy(obuf, acc.at[sl])
    pltpu.sync_copy(degs.at[0].at[sl], d0)
    pltpu.sync_copy(degs.at[1].at[sl], d1)
    pltpu.sync_copy(g1_hbm.at[sl], vg1)

    def frow(i4, _):
        for k in range(4):
            i = i4 * 4 + k
            no = _qrsqrt(d0[i] + d1[i] + 1.0)
            obuf[i] = vg1[i] * no
        return 0
    lax.fori_loop(0, STR // 4, frow, 0)
    pltpu.sync_copy(obuf, ftab.at[sl])
    pltpu.sync_copy(srcp.at[wid], sidx)
    pltpu.sync_copy(dstp.at[wid], didx)
    plsc.subcore_barrier()
    _scat_edge_loop(sidx, didx, rows, ftab, acc, sems[:NBUF], sems[NBUF:])
    plsc.subcore_barrier()
    pltpu.sync_copy(acc.at[sl], obuf)
    pltpu.sync_copy(obuf, agg_out.at[c].at[sl])


@functools.partial(
    pl.kernel,
    out_type=jax.ShapeDtypeStruct((NC, NP, HP), jnp.float32),
    mesh=_mesh,
    scratch_types=[
        pltpu.VMEM((STR, HP), jnp.float32),
        pltpu.VMEM((STR, HP), jnp.float32),
        pltpu.VMEM((STR, HP), jnp.float32),
        pltpu.VMEM((STR, HP), jnp.float32),
        pltpu.VMEM((STR, HP), jnp.float32),
        pltpu.VMEM((STR, HP), jnp.float32),
        pltpu.VMEM((1, HP), jnp.float32),
    ] + _scat_scratch,
    compiler_params=_sc_params,
)
def _scat2(agg1p, g1_hbm, degs, degd, b1_hbm, srcp, dstp, agg_out,
           d0, d1, e0, e1, vg1, a0, v_b1,
           sidx, didx, rows, obuf, ftab, acc, *sems):
    c = lax.axis_index("c")
    s = lax.axis_index("s")
    wid = c * NS + s
    sl = pl.ds(s * STR, STR)
    _zero_rows(obuf, STR)
    pltpu.sync_copy(obuf, acc.at[sl])
    pltpu.sync_copy(degs.at[0].at[sl], d0)
    pltpu.sync_copy(degs.at[1].at[sl], d1)
    pltpu.sync_copy(degd.at[0].at[sl], e0)
    pltpu.sync_copy(degd.at[1].at[sl], e1)
    pltpu.sync_copy(g1_hbm.at[sl], vg1)
    pltpu.sync_copy(agg1p.at[0].at[sl], a0)
    pltpu.sync_copy(b1_hbm, v_b1)
    b1v = v_b1[0]
    zero = jnp.zeros((HP,), jnp.float32)
    base = s * STR

    def nrow(i4, _):
        for k in range(4):
            i = i4 * 4 + k
            no = _qrsqrt(d0[i] + d1[i] + 1.0)
            ni = _qrsqrt(e0[i] + e1[i] + 1.0)
            hi = base + i >= N
            d0[i] = jnp.where(hi, zero, no)
            e0[i] = jnp.where(hi, zero, ni)
        return 0
    lax.fori_loop(0, STR // 4, nrow, 0)
    pltpu.sync_copy(agg1p.at[1].at[sl], d1)

    def frow(i4, _):
        for k in range(4):
            i = i4 * 4 + k
            no = d0[i]
            f1 = vg1[i] * no
            a = a0[i] + d1[i] + f1
            h1 = jnp.maximum(a * e0[i] + b1v, 0.0)
            obuf[i] = h1 * no
        return 0
    lax.fori_loop(0, STR // 4, frow, 0)
    pltpu.sync_copy(obuf, ftab.at[sl])
    pltpu.sync_copy(srcp.at[wid], sidx)
    pltpu.sync_copy(dstp.at[wid], didx)
    plsc.subcore_barrier()
    _scat_edge_loop(sidx, didx, rows, ftab, acc, sems[:NBUF], sems[NBUF:])
    plsc.subcore_barrier()
    pltpu.sync_copy(acc.at[sl], obuf)
    pltpu.sync_copy(obuf, agg_out.at[c].at[sl])


def _mm1_body(h_ref, w_ref, o_ref):
    o_ref[...] = jnp.dot(h_ref[...], w_ref[...], preferred_element_type=jnp.float32)


def _mm1(h_pad, w1p):
    return pl.pallas_call(
        _mm1_body,
        grid=(NP // RB,),
        in_specs=[
            pl.BlockSpec((RB, IN_DIM), lambda i: (i, 0)),
            pl.BlockSpec((IN_DIM, HP), lambda i: (0, 0)),
        ],
        out_specs=pl.BlockSpec((RB, HP), lambda i: (i, 0)),
        out_shape=jax.ShapeDtypeStruct((NP, HP), jnp.float32),
    )(h_pad, w1p)


def _out_body(degs_ref, degd_ref, g1_ref, agg1_ref, agg2_ref, b1_ref, b2_ref,
              w2_ref, wr_ref, br_ref, o_ref):
    i = pl.program_id(0)
    rows = lax.broadcasted_iota(jnp.int32, (RB, HP), 0) + i * RB
    valid = rows < N
    no = jnp.where(valid, lax.rsqrt(degs_ref[0] + degs_ref[1] + 1.0), 0.0)
    ni = jnp.where(valid, lax.rsqrt(degd_ref[0] + degd_ref[1] + 1.0), 0.0)
    f1 = g1_ref[...] * no
    a1 = agg1_ref[0] + agg1_ref[1] + f1
    h1 = jnp.maximum(a1 * ni + b1_ref[...], 0.0)
    f2 = h1 * no
    a2 = agg2_ref[0] + agg2_ref[1] + f2
    h2 = (jnp.dot(a2, w2_ref[...], preferred_element_type=jnp.float32)
          * ni + b2_ref[...])
    hr = jnp.maximum(h2, 0.0)
    o_ref[...] = jnp.dot(hr, wr_ref[...], preferred_element_type=jnp.float32) + br_ref[...]


def _outk(degs, degd, g1, agg1, agg2, b1p, b2p, w2p, wrp, brp):
    spec2 = pl.BlockSpec((NC, RB, HP), lambda i: (0, i, 0))
    spec = pl.BlockSpec((RB, HP), lambda i: (i, 0))
    bspec = pl.BlockSpec((1, HP), lambda i: (0, 0))
    return pl.pallas_call(
        _out_body,
        grid=(NP // RB,),
        in_specs=[
            spec2, spec2, spec, spec2, spec2, bspec, bspec,
            pl.BlockSpec((HP, HP), lambda i: (0, 0)),
            pl.BlockSpec((HP, IN_DIM), lambda i: (0, 0)),
            pl.BlockSpec((1, IN_DIM), lambda i: (0, 0)),
        ],
        out_specs=pl.BlockSpec((RB, IN_DIM), lambda i: (i, 0)),
        out_shape=jax.ShapeDtypeStruct((NP, IN_DIM), jnp.float32),
    )(degs, degd, g1, agg1, agg2, b1p, b2p, w2p, wrp, brp)


def kernel(h, edge_index, W1, b1, W2, b2, Wr, br):
    src = edge_index[0].astype(jnp.int32)
    dst = edge_index[1].astype(jnp.int32)
    pad = jnp.full((EPAD - E,), N, jnp.int32)
    srcp = jnp.concatenate([src, pad]).reshape(NW, CH, 128)
    dstp = jnp.concatenate([dst, pad]).reshape(NW, CH, 128)
    h_pad = jnp.pad(h, ((0, NP - N), (0, 0)))
    w1p = jnp.pad(W1, ((0, 0), (0, HP - HID)))
    w2p = jnp.pad(W2, ((0, HP - HID), (0, HP - HID)))
    wrp = jnp.pad(Wr, ((0, HP - HID), (0, 0)))
    b1p = jnp.pad(b1, (0, HP - HID)).reshape(1, HP)
    b2p = jnp.pad(b2, (0, HP - HID)).reshape(1, HP)
    brp = br.reshape(1, IN_DIM)

    g1 = _mm1(h_pad, w1p)
    degs, degd = _hist(srcp, dstp)
    agg1 = _scat1(g1, degs, srcp, dstp)
    agg2 = _scat2(agg1, g1, degs, degd, b1p, srcp, dstp)
    out = _outk(degs, degd, g1, agg1, agg2, b1p, b2p, w2p, wrp, brp)
    return out[:N]


# speedup vs baseline: 28.6314x; 1.1871x over previous
import functools

import jax
import jax.numpy as jnp
from jax import lax
from jax.experimental import pallas as pl
from jax.experimental.pallas import tpu as pltpu
from jax.experimental.pallas import tpu_sc as plsc

N = 10000
E = 320000
IN_DIM = 128
HID = 8
HP = 16
NP = 10240
NC, NS = 2, 16
NW = NC * NS
EROWS = 2500
CH = 78
TR = NW * CH
NTAIL = EROWS - TR
STR = NP // NS
RB = 1280
NBUF = 6
HG = 6

_mesh = plsc.VectorSubcoreMesh(core_axis_name="c", subcore_axis_name="s")
_sc_params = pltpu.CompilerParams(use_tc_tiling_on_sc=False)


def _zero_rows(buf, n_rows):
    def body(i, _):
        buf[i] = jnp.zeros((HP,), jnp.float32)
        return 0
    lax.fori_loop(0, n_rows, body, 0)


def _qrsqrt(x):
    i = lax.bitcast_convert_type(x, jnp.int32)
    y = lax.bitcast_convert_type(jnp.int32(0x5F3759DF) - (i >> 1), jnp.float32)
    for _ in range(3):
        y = y * (1.5 - 0.5 * x * y * y)
    return y


@functools.partial(
    pl.kernel,
    out_type=[
        jax.ShapeDtypeStruct((NC, NP, HP), jnp.float32),
        jax.ShapeDtypeStruct((NC, NP, HP), jnp.float32),
    ],
    mesh=_mesh,
    scratch_types=[
        pltpu.VMEM((CH, 128), jnp.int32),
        pltpu.VMEM((CH, 128), jnp.int32),
        pltpu.VMEM((128, HP), jnp.float32),
        pltpu.VMEM((STR, HP), jnp.float32),
        pltpu.VMEM_SHARED((NP, HP), jnp.float32),
        pltpu.VMEM_SHARED((NP, HP), jnp.float32),
        pltpu.SemaphoreType.DMA,
        pltpu.SemaphoreType.DMA,
    ],
    compiler_params=_sc_params,
)
def _hist(e2, degs_out, degd_out, sidx, didx, ones, obuf, acc_s, acc_d,
          sem_s, sem_d):
    c = lax.axis_index("c")
    s = lax.axis_index("s")
    wid = c * NS + s
    _zero_rows(obuf, STR)
    pltpu.sync_copy(obuf, acc_s.at[pl.ds(s * STR, STR)])
    pltpu.sync_copy(obuf, acc_d.at[pl.ds(s * STR, STR)])

    def fill_ones(i, _):
        ones[i] = jnp.full((HP,), 1.0, jnp.float32)
        return 0
    lax.fori_loop(0, 128, fill_ones, 0)
    pltpu.sync_copy(e2.at[0].at[pl.ds(wid * CH, CH)], sidx)
    pltpu.sync_copy(e2.at[1].at[pl.ds(wid * CH, CH)], didx)
    plsc.subcore_barrier()

    def group(gi, _):
        handles = []
        for k in range(HG):
            j = gi * HG + k
            handles.append(
                pltpu.async_copy(ones, acc_s.at[sidx.at[j]], sem_s, add=True))
            handles.append(
                pltpu.async_copy(ones, acc_d.at[didx.at[j]], sem_d, add=True))
        for hdl in handles:
            hdl.wait()
        return 0
    lax.fori_loop(0, CH // HG, group, 0)

    @pl.when(wid < NTAIL)
    def _():
        pltpu.sync_copy(e2.at[0].at[TR + wid], sidx.at[0])
        pltpu.sync_copy(e2.at[1].at[TR + wid], didx.at[0])
        pltpu.sync_copy(ones, acc_s.at[sidx.at[0]], add=True)
        pltpu.sync_copy(ones, acc_d.at[didx.at[0]], add=True)

    plsc.subcore_barrier()
    pltpu.sync_copy(acc_s.at[pl.ds(s * STR, STR)], obuf)
    pltpu.sync_copy(obuf, degs_out.at[c].at[pl.ds(s * STR, STR)])
    pltpu.sync_copy(acc_d.at[pl.ds(s * STR, STR)], obuf)
    pltpu.sync_copy(obuf, degd_out.at[c].at[pl.ds(s * STR, STR)])


def _scat_edge_loop(e2, wid, sidx, didx, rows, ftab, acc, gsems, ssems):
    for b in range(NBUF - 1):
        pltpu.async_copy(ftab.at[sidx.at[b]], rows.at[b], gsems[b])

    def group(gi, _):
        for b in range(NBUF):
            j = gi * NBUF + b
            jn = j + (NBUF - 1)
            bn = (b + NBUF - 1) % NBUF

            @pl.when(jn < CH)
            def _():
                @pl.when(j >= 1)
                def _():
                    pltpu.make_async_copy(
                        rows.at[bn], acc.at[didx.at[jn - NBUF]], ssems[bn]
                    ).wait()
                pltpu.async_copy(ftab.at[sidx.at[jn]], rows.at[bn], gsems[bn])

            pltpu.make_async_copy(ftab.at[sidx.at[j]], rows.at[b], gsems[b]).wait()
            pltpu.async_copy(rows.at[b], acc.at[didx.at[j]], ssems[b], add=True)
        return 0
    lax.fori_loop(0, CH // NBUF, group, 0)
    for b in range(NBUF):
        pltpu.make_async_copy(
            rows.at[b], acc.at[didx.at[CH - NBUF + b]], ssems[b]).wait()

    @pl.when(wid < NTAIL)
    def _():
        pltpu.sync_copy(e2.at[0].at[TR + wid], sidx.at[0])
        pltpu.sync_copy(e2.at[1].at[TR + wid], didx.at[0])
        pltpu.sync_copy(ftab.at[sidx.at[0]], rows.at[0])
        pltpu.sync_copy(rows.at[0], acc.at[didx.at[0]], add=True)


_scat_scratch = [
    pltpu.VMEM((CH, 128), jnp.int32),
    pltpu.VMEM((CH, 128), jnp.int32),
    pltpu.VMEM((NBUF, 128, HP), jnp.float32),
    pltpu.VMEM((STR, HP), jnp.float32),
    pltpu.VMEM_SHARED((NP, HP), jnp.float32),
    pltpu.VMEM_SHARED((NP, HP), jnp.float32),
] + [pltpu.SemaphoreType.DMA] * (2 * NBUF)


@functools.partial(
    pl.kernel,
    out_type=jax.ShapeDtypeStruct((NC, NP, HP), jnp.float32),
    mesh=_mesh,
    scratch_types=[
        pltpu.VMEM((STR, HP), jnp.float32),
        pltpu.VMEM((STR, HP), jnp.float32),
        pltpu.VMEM((STR, HP), jnp.float32),
    ] + _scat_scratch,
    compiler_params=_sc_params,
)
def _scat1(g1_hbm, degs, e2, agg_out,
           d0, d1, vg1, sidx, didx, rows, obuf, ftab, acc, *sems):
    c = lax.axis_index("c")
    s = lax.axis_index("s")
    wid = c * NS + s
    sl = pl.ds(s * STR, STR)
    _zero_rows(obuf, STR)
    pltpu.sync_copy(obuf, acc.at[sl])
    pltpu.sync_copy(degs.at[0].at[sl], d0)
    pltpu.sync_copy(degs.at[1].at[sl], d1)
    pltpu.sync_copy(g1_hbm.at[sl], vg1)

    def frow(i4, _):
        for k in range(4):
            i = i4 * 4 + k
            no = _qrsqrt(d0[i] + d1[i] + 1.0)
            obuf[i] = vg1[i] * no
        return 0
    lax.fori_loop(0, STR // 4, frow, 0)
    pltpu.sync_copy(obuf, ftab.at[sl])
    pltpu.sync_copy(e2.at[0].at[pl.ds(wid * CH, CH)], sidx)
    pltpu.sync_copy(e2.at[1].at[pl.ds(wid * CH, CH)], didx)
    plsc.subcore_barrier()
    _scat_edge_loop(e2, wid, sidx, didx, rows, ftab, acc, sems[:NBUF], sems[NBUF:])
    plsc.subcore_barrier()
    pltpu.sync_copy(acc.at[sl], obuf)
    pltpu.sync_copy(obuf, agg_out.at[c].at[sl])


@functools.partial(
    pl.kernel,
    out_type=jax.ShapeDtypeStruct((NC, NP, HP), jnp.float32),
    mesh=_mesh,
    scratch_types=[
        pltpu.VMEM((STR, HP), jnp.float32),
        pltpu.VMEM((STR, HP), jnp.float32),
        pltpu.VMEM((STR, HP), jnp.float32),
        pltpu.VMEM((STR, HP), jnp.float32),
        pltpu.VMEM((STR, HP), jnp.float32),
        pltpu.VMEM((STR, HP), jnp.float32),
        pltpu.VMEM((1, HP), jnp.float32),
    ] + _scat_scratch,
    compiler_params=_sc_params,
)
def _scat2(agg1p, g1_hbm, degs, degd, b1_hbm, e2, agg_out,
           d0, d1, e0, e1, vg1, a0, v_b1,
           sidx, didx, rows, obuf, ftab, acc, *sems):
    c = lax.axis_index("c")
    s = lax.axis_index("s")
    wid = c * NS + s
    sl = pl.ds(s * STR, STR)
    _zero_rows(obuf, STR)
    pltpu.sync_copy(obuf, acc.at[sl])
    pltpu.sync_copy(degs.at[0].at[sl], d0)
    pltpu.sync_copy(degs.at[1].at[sl], d1)
    pltpu.sync_copy(degd.at[0].at[sl], e0)
    pltpu.sync_copy(degd.at[1].at[sl], e1)
    pltpu.sync_copy(g1_hbm.at[sl], vg1)
    pltpu.sync_copy(agg1p.at[0].at[sl], a0)
    pltpu.sync_copy(b1_hbm, v_b1)
    b1v = v_b1[0]
    zero = jnp.zeros((HP,), jnp.float32)
    base = s * STR

    def nrow(i4, _):
        for k in range(4):
            i = i4 * 4 + k
            no = _qrsqrt(d0[i] + d1[i] + 1.0)
            ni = _qrsqrt(e0[i] + e1[i] + 1.0)
            hi = base + i >= N
            d0[i] = jnp.where(hi, zero, no)
            e0[i] = jnp.where(hi, zero, ni)
        return 0
    lax.fori_loop(0, STR // 4, nrow, 0)
    pltpu.sync_copy(agg1p.at[1].at[sl], d1)

    def frow(i4, _):
        for k in range(4):
            i = i4 * 4 + k
            no = d0[i]
            f1 = vg1[i] * no
            a = a0[i] + d1[i] + f1
            h1 = jnp.maximum(a * e0[i] + b1v, 0.0)
            obuf[i] = h1 * no
        return 0
    lax.fori_loop(0, STR // 4, frow, 0)
    pltpu.sync_copy(obuf, ftab.at[sl])
    pltpu.sync_copy(e2.at[0].at[pl.ds(wid * CH, CH)], sidx)
    pltpu.sync_copy(e2.at[1].at[pl.ds(wid * CH, CH)], didx)
    plsc.subcore_barrier()
    _scat_edge_loop(e2, wid, sidx, didx, rows, ftab, acc, sems[:NBUF], sems[NBUF:])
    plsc.subcore_barrier()
    pltpu.sync_copy(acc.at[sl], obuf)
    pltpu.sync_copy(obuf, agg_out.at[c].at[sl])


def _mm1_body(h_ref, w_ref, o_ref):
    i = pl.program_id(0)
    rows = lax.broadcasted_iota(jnp.int32, (RB, HP), 0) + i * RB
    w = jnp.concatenate(
        [w_ref[...], jnp.zeros((IN_DIM, HP - HID), jnp.float32)], axis=1)
    g = jnp.dot(h_ref[...], w, preferred_element_type=jnp.float32)
    o_ref[...] = jnp.where(rows < N, g, 0.0)


def _mm1(h, w1):
    return pl.pallas_call(
        _mm1_body,
        grid=(NP // RB,),
        in_specs=[
            pl.BlockSpec((RB, IN_DIM), lambda i: (i, 0)),
            pl.BlockSpec((IN_DIM, HID), lambda i: (0, 0)),
        ],
        out_specs=pl.BlockSpec((RB, HP), lambda i: (i, 0)),
        out_shape=jax.ShapeDtypeStruct((NP, HP), jnp.float32),
    )(h, w1)


def _out_body(degs_ref, degd_ref, g1_ref, agg1_ref, agg2_ref, b1_ref, b2_ref,
              w2_ref, wr_ref, br_ref, o_ref):
    i = pl.program_id(0)
    rows = lax.broadcasted_iota(jnp.int32, (RB, HP), 0) + i * RB
    valid = rows < N
    no = jnp.where(valid, lax.rsqrt(degs_ref[0] + degs_ref[1] + 1.0), 0.0)
    ni = jnp.where(valid, lax.rsqrt(degd_ref[0] + degd_ref[1] + 1.0), 0.0)
    b1 = jnp.concatenate(
        [b1_ref[...], jnp.zeros((1, HP - HID), jnp.float32)], axis=1)
    b2 = jnp.concatenate(
        [b2_ref[...], jnp.zeros((1, HP - HID), jnp.float32)], axis=1)
    w2 = jnp.concatenate(
        [w2_ref[...], jnp.zeros((HP - HID, HID), jnp.float32)], axis=0)
    w2 = jnp.concatenate([w2, jnp.zeros((HP, HP - HID), jnp.float32)], axis=1)
    wr = jnp.concatenate(
        [wr_ref[...], jnp.zeros((HP - HID, IN_DIM), jnp.float32)], axis=0)
    f1 = g1_ref[...] * no
    a1 = agg1_ref[0] + agg1_ref[1] + f1
    h1 = jnp.maximum(a1 * ni + b1, 0.0)
    f2 = h1 * no
    a2 = agg2_ref[0] + agg2_ref[1] + f2
    h2 = (jnp.dot(a2, w2, preferred_element_type=jnp.float32)
          * ni + b2)
    hr = jnp.maximum(h2, 0.0)
    o_ref[...] = jnp.dot(hr, wr, preferred_element_type=jnp.float32) + br_ref[...]


def _outk(degs, degd, g1, agg1, agg2, b1, b2, W2, Wr, br):
    spec2 = pl.BlockSpec((NC, RB, HP), lambda i: (0, i, 0))
    spec = pl.BlockSpec((RB, HP), lambda i: (i, 0))
    bspec = pl.BlockSpec((1, HID), lambda i: (0, 0))
    return pl.pallas_call(
        _out_body,
        grid=(NP // RB,),
        in_specs=[
            spec2, spec2, spec, spec2, spec2, bspec, bspec,
            pl.BlockSpec((HID, HID), lambda i: (0, 0)),
            pl.BlockSpec((HID, IN_DIM), lambda i: (0, 0)),
            pl.BlockSpec((1, IN_DIM), lambda i: (0, 0)),
        ],
        out_specs=pl.BlockSpec((RB, IN_DIM), lambda i: (i, 0)),
        out_shape=jax.ShapeDtypeStruct((N, IN_DIM), jnp.float32),
    )(degs, degd, g1, agg1, agg2, b1, b2, W2, Wr, br)


def kernel(h, edge_index, W1, b1, W2, b2, Wr, br):
    e2 = edge_index.astype(jnp.int32).reshape(2, EROWS, 128)
    b1r = b1.reshape(1, HID)
    b2r = b2.reshape(1, HID)
    brr = br.reshape(1, IN_DIM)
    b1p = jnp.pad(b1, (0, HP - HID)).reshape(1, HP)

    g1 = _mm1(h, W1)
    degs, degd = _hist(e2)
    agg1 = _scat1(g1, degs, e2)
    agg2 = _scat2(agg1, g1, degs, degd, b1p, e2)
    return _outk(degs, degd, g1, agg1, agg2, b1r, b2r, W2, Wr, brr)


# speedup vs baseline: 28.6554x; 1.0008x over previous
import functools

import jax
import jax.numpy as jnp
from jax import lax
from jax.experimental import pallas as pl
from jax.experimental.pallas import tpu as pltpu
from jax.experimental.pallas import tpu_sc as plsc

N = 10000
E = 320000
IN_DIM = 128
HID = 8
HP = 16
NP = 10240
NC, NS = 2, 16
NW = NC * NS
EROWS = 2500
CH = 78
TR = NW * CH
NTAIL = EROWS - TR
STR = NP // NS
RB = 1280
NBUF = 6
HG = 13

_mesh = plsc.VectorSubcoreMesh(core_axis_name="c", subcore_axis_name="s")
_sc_params = pltpu.CompilerParams(use_tc_tiling_on_sc=False)


def _zero_rows(buf, n_rows):
    def body(i, _):
        buf[i] = jnp.zeros((HP,), jnp.float32)
        return 0
    lax.fori_loop(0, n_rows, body, 0)


def _qrsqrt(x):
    i = lax.bitcast_convert_type(x, jnp.int32)
    y = lax.bitcast_convert_type(jnp.int32(0x5F3759DF) - (i >> 1), jnp.float32)
    for _ in range(3):
        y = y * (1.5 - 0.5 * x * y * y)
    return y


@functools.partial(
    pl.kernel,
    out_type=[
        jax.ShapeDtypeStruct((NC, NP, HP), jnp.float32),
        jax.ShapeDtypeStruct((NC, NP, HP), jnp.float32),
    ],
    mesh=_mesh,
    scratch_types=[
        pltpu.VMEM((CH, 128), jnp.int32),
        pltpu.VMEM((CH, 128), jnp.int32),
        pltpu.VMEM((128, HP), jnp.float32),
        pltpu.VMEM((STR, HP), jnp.float32),
        pltpu.VMEM_SHARED((NP, HP), jnp.float32),
        pltpu.VMEM_SHARED((NP, HP), jnp.float32),
        pltpu.SemaphoreType.DMA,
        pltpu.SemaphoreType.DMA,
    ],
    compiler_params=_sc_params,
)
def _hist(e2, degs_out, degd_out, sidx, didx, ones, obuf, acc_s, acc_d,
          sem_s, sem_d):
    c = lax.axis_index("c")
    s = lax.axis_index("s")
    wid = c * NS + s
    _zero_rows(obuf, STR)
    pltpu.sync_copy(obuf, acc_s.at[pl.ds(s * STR, STR)])
    pltpu.sync_copy(obuf, acc_d.at[pl.ds(s * STR, STR)])

    def fill_ones(i, _):
        ones[i] = jnp.full((HP,), 1.0, jnp.float32)
        return 0
    lax.fori_loop(0, 128, fill_ones, 0)
    pltpu.sync_copy(e2.at[0].at[pl.ds(wid * CH, CH)], sidx)
    pltpu.sync_copy(e2.at[1].at[pl.ds(wid * CH, CH)], didx)
    plsc.subcore_barrier()

    def group(gi, _):
        handles = []
        for k in range(HG):
            j = gi * HG + k
            handles.append(
                pltpu.async_copy(ones, acc_s.at[sidx.at[j]], sem_s, add=True))
            handles.append(
                pltpu.async_copy(ones, acc_d.at[didx.at[j]], sem_d, add=True))
        for hdl in handles:
            hdl.wait()
        return 0
    lax.fori_loop(0, CH // HG, group, 0)

    @pl.when(wid < NTAIL)
    def _():
        pltpu.sync_copy(e2.at[0].at[TR + wid], sidx.at[0])
        pltpu.sync_copy(e2.at[1].at[TR + wid], didx.at[0])
        pltpu.sync_copy(ones, acc_s.at[sidx.at[0]], add=True)
        pltpu.sync_copy(ones, acc_d.at[didx.at[0]], add=True)

    plsc.subcore_barrier()
    pltpu.sync_copy(acc_s.at[pl.ds(s * STR, STR)], obuf)
    pltpu.sync_copy(obuf, degs_out.at[c].at[pl.ds(s * STR, STR)])
    pltpu.sync_copy(acc_d.at[pl.ds(s * STR, STR)], obuf)
    pltpu.sync_copy(obuf, degd_out.at[c].at[pl.ds(s * STR, STR)])


def _scat_edge_loop(e2, wid, sidx, didx, rows, ftab, acc, gsems, ssems):
    for b in range(NBUF - 1):
        pltpu.async_copy(ftab.at[sidx.at[b]], rows.at[b], gsems[b])

    def group(gi, _):
        for b in range(NBUF):
            j = gi * NBUF + b
            jn = j + (NBUF - 1)
            bn = (b + NBUF - 1) % NBUF

            @pl.when(jn < CH)
            def _():
                @pl.when(j >= 1)
                def _():
                    pltpu.make_async_copy(
                        rows.at[bn], acc.at[didx.at[jn - NBUF]], ssems[bn]
                    ).wait()
                pltpu.async_copy(ftab.at[sidx.at[jn]], rows.at[bn], gsems[bn])

            pltpu.make_async_copy(ftab.at[sidx.at[j]], rows.at[b], gsems[b]).wait()
            pltpu.async_copy(rows.at[b], acc.at[didx.at[j]], ssems[b], add=True)
        return 0
    lax.fori_loop(0, CH // NBUF, group, 0)
    for b in range(NBUF):
        pltpu.make_async_copy(
            rows.at[b], acc.at[didx.at[CH - NBUF + b]], ssems[b]).wait()

    @pl.when(wid < NTAIL)
    def _():
        pltpu.sync_copy(e2.at[0].at[TR + wid], sidx.at[0])
        pltpu.sync_copy(e2.at[1].at[TR + wid], didx.at[0])
        pltpu.sync_copy(ftab.at[sidx.at[0]], rows.at[0])
        pltpu.sync_copy(rows.at[0], acc.at[didx.at[0]], add=True)


_scat_scratch = [
    pltpu.VMEM((CH, 128), jnp.int32),
    pltpu.VMEM((CH, 128), jnp.int32),
    pltpu.VMEM((NBUF, 128, HP), jnp.float32),
    pltpu.VMEM((STR, HP), jnp.float32),
    pltpu.VMEM_SHARED((NP, HP), jnp.float32),
    pltpu.VMEM_SHARED((NP, HP), jnp.float32),
] + [pltpu.SemaphoreType.DMA] * (2 * NBUF)


@functools.partial(
    pl.kernel,
    out_type=jax.ShapeDtypeStruct((NC, NP, HP), jnp.float32),
    mesh=_mesh,
    scratch_types=[
        pltpu.VMEM((STR, HP), jnp.float32),
        pltpu.VMEM((STR, HP), jnp.float32),
        pltpu.VMEM((STR, HP), jnp.float32),
    ] + _scat_scratch,
    compiler_params=_sc_params,
)
def _scat1(g1_hbm, degs, e2, agg_out,
           d0, d1, vg1, sidx, didx, rows, obuf, ftab, acc, *sems):
    c = lax.axis_index("c")
    s = lax.axis_index("s")
    wid = c * NS + s
    sl = pl.ds(s * STR, STR)
    _zero_rows(obuf, STR)
    pltpu.sync_copy(obuf, acc.at[sl])
    pltpu.sync_copy(degs.at[0].at[sl], d0)
    pltpu.sync_copy(degs.at[1].at[sl], d1)
    pltpu.sync_copy(g1_hbm.at[sl], vg1)

    def frow(i4, _):
        for k in range(4):
            i = i4 * 4 + k
            no = _qrsqrt(d0[i] + d1[i] + 1.0)
            obuf[i] = vg1[i] * no
        return 0
    lax.fori_loop(0, STR // 4, frow, 0)
    pltpu.sync_copy(obuf, ftab.at[sl])
    pltpu.sync_copy(e2.at[0].at[pl.ds(wid * CH, CH)], sidx)
    pltpu.sync_copy(e2.at[1].at[pl.ds(wid * CH, CH)], didx)
    plsc.subcore_barrier()
    _scat_edge_loop(e2, wid, sidx, didx, rows, ftab, acc, sems[:NBUF], sems[NBUF:])
    plsc.subcore_barrier()
    pltpu.sync_copy(acc.at[sl], obuf)
    pltpu.sync_copy(obuf, agg_out.at[c].at[sl])


@functools.partial(
    pl.kernel,
    out_type=jax.ShapeDtypeStruct((NC, NP, HP), jnp.float32),
    mesh=_mesh,
    scratch_types=[
        pltpu.VMEM((STR, HP), jnp.float32),
        pltpu.VMEM((STR, HP), jnp.float32),
        pltpu.VMEM((STR, HP), jnp.float32),
        pltpu.VMEM((STR, HP), jnp.float32),
        pltpu.VMEM((STR, HP), jnp.float32),
        pltpu.VMEM((STR, HP), jnp.float32),
        pltpu.VMEM((1, HP), jnp.float32),
    ] + _scat_scratch,
    compiler_params=_sc_params,
)
def _scat2(agg1p, g1_hbm, degs, degd, b1_hbm, e2, agg_out,
           d0, d1, e0, e1, vg1, a0, v_b1,
           sidx, didx, rows, obuf, ftab, acc, *sems):
    c = lax.axis_index("c")
    s = lax.axis_index("s")
    wid = c * NS + s
    sl = pl.ds(s * STR, STR)
    _zero_rows(obuf, STR)
    pltpu.sync_copy(obuf, acc.at[sl])
    pltpu.sync_copy(degs.at[0].at[sl], d0)
    pltpu.sync_copy(degs.at[1].at[sl], d1)
    pltpu.sync_copy(degd.at[0].at[sl], e0)
    pltpu.sync_copy(degd.at[1].at[sl], e1)
    pltpu.sync_copy(g1_hbm.at[sl], vg1)
    pltpu.sync_copy(agg1p.at[0].at[sl], a0)
    pltpu.sync_copy(b1_hbm, v_b1)
    b1v = v_b1[0]
    zero = jnp.zeros((HP,), jnp.float32)
    base = s * STR

    def nrow(i4, _):
        for k in range(4):
            i = i4 * 4 + k
            no = _qrsqrt(d0[i] + d1[i] + 1.0)
            ni = _qrsqrt(e0[i] + e1[i] + 1.0)
            hi = base + i >= N
            d0[i] = jnp.where(hi, zero, no)
            e0[i] = jnp.where(hi, zero, ni)
        return 0
    lax.fori_loop(0, STR // 4, nrow, 0)
    pltpu.sync_copy(agg1p.at[1].at[sl], d1)

    def frow(i4, _):
        for k in range(4):
            i = i4 * 4 + k
            no = d0[i]
            f1 = vg1[i] * no
            a = a0[i] + d1[i] + f1
            h1 = jnp.maximum(a * e0[i] + b1v, 0.0)
            obuf[i] = h1 * no
        return 0
    lax.fori_loop(0, STR // 4, frow, 0)
    pltpu.sync_copy(obuf, ftab.at[sl])
    pltpu.sync_copy(e2.at[0].at[pl.ds(wid * CH, CH)], sidx)
    pltpu.sync_copy(e2.at[1].at[pl.ds(wid * CH, CH)], didx)
    plsc.subcore_barrier()
    _scat_edge_loop(e2, wid, sidx, didx, rows, ftab, acc, sems[:NBUF], sems[NBUF:])
    plsc.subcore_barrier()
    pltpu.sync_copy(acc.at[sl], obuf)
    pltpu.sync_copy(obuf, agg_out.at[c].at[sl])


def _mm1_body(h_ref, w_ref, o_ref):
    i = pl.program_id(0)
    rows = lax.broadcasted_iota(jnp.int32, (RB, HP), 0) + i * RB
    w = jnp.concatenate(
        [w_ref[...], jnp.zeros((IN_DIM, HP - HID), jnp.float32)], axis=1)
    g = jnp.dot(h_ref[...], w, preferred_element_type=jnp.float32)
    o_ref[...] = jnp.where(rows < N, g, 0.0)


def _mm1(h, w1):
    return pl.pallas_call(
        _mm1_body,
        grid=(NP // RB,),
        in_specs=[
            pl.BlockSpec((RB, IN_DIM), lambda i: (i, 0)),
            pl.BlockSpec((IN_DIM, HID), lambda i: (0, 0)),
        ],
        out_specs=pl.BlockSpec((RB, HP), lambda i: (i, 0)),
        out_shape=jax.ShapeDtypeStruct((NP, HP), jnp.float32),
    )(h, w1)


def _out_body(degs_ref, degd_ref, g1_ref, agg1_ref, agg2_ref, b1_ref, b2_ref,
              w2_ref, wr_ref, br_ref, o_ref):
    i = pl.program_id(0)
    rows = lax.broadcasted_iota(jnp.int32, (RB, HP), 0) + i * RB
    valid = rows < N
    no = jnp.where(valid, lax.rsqrt(degs_ref[0] + degs_ref[1] + 1.0), 0.0)
    ni = jnp.where(valid, lax.rsqrt(degd_ref[0] + degd_ref[1] + 1.0), 0.0)
    b1 = jnp.concatenate(
        [b1_ref[...], jnp.zeros((1, HP - HID), jnp.float32)], axis=1)
    b2 = jnp.concatenate(
        [b2_ref[...], jnp.zeros((1, HP - HID), jnp.float32)], axis=1)
    w2 = jnp.concatenate(
        [w2_ref[...], jnp.zeros((HP - HID, HID), jnp.float32)], axis=0)
    w2 = jnp.concatenate([w2, jnp.zeros((HP, HP - HID), jnp.float32)], axis=1)
    wr = jnp.concatenate(
        [wr_ref[...], jnp.zeros((HP - HID, IN_DIM), jnp.float32)], axis=0)
    f1 = g1_ref[...] * no
    a1 = agg1_ref[0] + agg1_ref[1] + f1
    h1 = jnp.maximum(a1 * ni + b1, 0.0)
    f2 = h1 * no
    a2 = agg2_ref[0] + agg2_ref[1] + f2
    h2 = (jnp.dot(a2, w2, preferred_element_type=jnp.float32)
          * ni + b2)
    hr = jnp.maximum(h2, 0.0)
    o_ref[...] = jnp.dot(hr, wr, preferred_element_type=jnp.float32) + br_ref[...]


def _outk(degs, degd, g1, agg1, agg2, b1, b2, W2, Wr, br):
    spec2 = pl.BlockSpec((NC, RB, HP), lambda i: (0, i, 0))
    spec = pl.BlockSpec((RB, HP), lambda i: (i, 0))
    bspec = pl.BlockSpec((1, HID), lambda i: (0, 0))
    return pl.pallas_call(
        _out_body,
        grid=(NP // RB,),
        in_specs=[
            spec2, spec2, spec, spec2, spec2, bspec, bspec,
            pl.BlockSpec((HID, HID), lambda i: (0, 0)),
            pl.BlockSpec((HID, IN_DIM), lambda i: (0, 0)),
            pl.BlockSpec((1, IN_DIM), lambda i: (0, 0)),
        ],
        out_specs=pl.BlockSpec((RB, IN_DIM), lambda i: (i, 0)),
        out_shape=jax.ShapeDtypeStruct((N, IN_DIM), jnp.float32),
    )(degs, degd, g1, agg1, agg2, b1, b2, W2, Wr, br)


def kernel(h, edge_index, W1, b1, W2, b2, Wr, br):
    e2 = edge_index.astype(jnp.int32).reshape(2, EROWS, 128)
    b1r = b1.reshape(1, HID)
    b2r = b2.reshape(1, HID)
    brr = br.reshape(1, IN_DIM)
    b1p = jnp.pad(b1, (0, HP - HID)).reshape(1, HP)

    g1 = _mm1(h, W1)
    degs, degd = _hist(e2)
    agg1 = _scat1(g1, degs, e2)
    agg2 = _scat2(agg1, g1, degs, degd, b1p, e2)
    return _outk(degs, degd, g1, agg1, agg2, b1r, b2r, W2, Wr, brr)
